# Initial kernel scaffold; baseline (speedup 1.0000x reference)
import jax
import jax.numpy as jnp
from jax.experimental import pallas as pl


def kernel(out_features, data_input, edge_index, params):
    raise NotImplementedError("write your pallas kernel here")


# speedup vs baseline: 3.7062x; 3.7062x over previous
import jax
import jax.numpy as jnp
from jax import lax
from jax.experimental import pallas as pl
from jax.experimental.pallas import tpu as pltpu
from jax.experimental.pallas import tpu_sc as plsc

_N = 10000
_E = 320000
_H = 128
_C = 6

_NS = 16
_B = 80
_EPS = _E // _NS
_CH = _EPS // _B
_NP = 10240
_RPT = _NP // _NS
_HH = _H // 2

_BN1 = 1000
_BN2 = 2000


def _make_agg(with_deg):
    mesh = plsc.VectorSubcoreMesh(core_axis_name="c", subcore_axis_name="s",
                                  num_cores=2, num_subcores=_NS)
    outs = [jax.ShapeDtypeStruct((_NP, _HH), jnp.float32)
            for _ in range(4)]
    scratch = [
        pltpu.VMEM((_CH, _B), jnp.int32),
        pltpu.VMEM((_CH, _B), jnp.int32),
        pltpu.VMEM((_B, _HH), jnp.float32),
        pltpu.VMEM_SHARED((_NP, _HH), jnp.float32),
        pltpu.SemaphoreType.DMA,
    ]
    if with_deg:
        outs.append(jax.ShapeDtypeStruct((_NP, 16), jnp.float32))
        scratch += [
            pltpu.VMEM((_B, 16), jnp.float32),
            pltpu.VMEM_SHARED((_NP, 16), jnp.float32),
        ]

    def _pass(sid, xref, yref, zfeat, sidx, didx, rows, acc, sem, degp):
        pltpu.sync_copy(zfeat, acc.at[pl.ds(sid * _RPT, _RPT)])
        if degp is not None:
            ones80, zdeg, degy, onesv, accd = degp
            pltpu.sync_copy(zdeg, accd.at[pl.ds(sid * _RPT, _RPT)])
            pltpu.sync_copy(ones80, onesv)
        plsc.subcore_barrier()

        def chunk(i, carry):
            pltpu.async_copy(xref.at[sidx.at[i]], rows, sem).wait()
            pltpu.sync_copy(rows, acc.at[didx.at[i]], add=True)
            if degp is not None:
                pltpu.sync_copy(degp[3], degp[4].at[didx.at[i]], add=True)
            return carry

        lax.fori_loop(0, _CH, chunk, 0)
        plsc.subcore_barrier()
        pltpu.sync_copy(acc.at[pl.ds(sid * _RPT, _RPT)],
                        yref.at[pl.ds(sid * _RPT, _RPT)])
        if degp is not None:
            pltpu.sync_copy(degp[4].at[pl.ds(sid * _RPT, _RPT)],
                            degp[2].at[pl.ds(sid * _RPT, _RPT)])

    def _core(sid, xa, xb, ya, yb, zfeat, sidx, didx, rows, acc, sem, degp):
        _pass(sid, xa, ya, zfeat, sidx, didx, rows, acc, sem, degp)
        plsc.subcore_barrier()
        _pass(sid, xb, yb, zfeat, sidx, didx, rows, acc, sem, None)

    if with_deg:
        def body(x0a, x0b, x1a, x1b, src2, dst2, zfeat, ones80, zdeg,
                 y0a, y0b, y1a, y1b, degy,
                 sidx, didx, rows, acc, sem, onesv, accd):
            cid = lax.axis_index("c")
            sid = lax.axis_index("s")
            pltpu.sync_copy(src2.at[sid], sidx)
            pltpu.sync_copy(dst2.at[sid], didx)

            @pl.when(cid == 0)
            def _():
                _core(sid, x0a, x0b, y0a, y0b, zfeat, sidx, didx, rows,
                      acc, sem, (ones80, zdeg, degy, onesv, accd))

            @pl.when(cid == 1)
            def _():
                _core(sid, x1a, x1b, y1a, y1b, zfeat, sidx, didx, rows,
                      acc, sem, None)
    else:
        def body(x0a, x0b, x1a, x1b, src2, dst2, zfeat,
                 y0a, y0b, y1a, y1b,
                 sidx, didx, rows, acc, sem):
            cid = lax.axis_index("c")
            sid = lax.axis_index("s")
            pltpu.sync_copy(src2.at[sid], sidx)
            pltpu.sync_copy(dst2.at[sid], didx)

            @pl.when(cid == 0)
            def _():
                _core(sid, x0a, x0b, y0a, y0b, zfeat, sidx, didx, rows,
                      acc, sem, None)

            @pl.when(cid == 1)
            def _():
                _core(sid, x1a, x1b, y1a, y1b, zfeat, sidx, didx, rows,
                      acc, sem, None)

    return pl.kernel(body, out_type=tuple(outs), mesh=mesh,
                     scratch_types=tuple(scratch),
                     compiler_params=pltpu.CompilerParams(
                         use_tc_tiling_on_sc=False))


def _agg_pair(fn, x0, x1, src2, dst2, extras):
    halves = (x0[:, :_HH], x0[:, _HH:], x1[:, :_HH], x1[:, _HH:])
    res = fn(*halves, src2, dst2, *extras)
    y0 = jnp.concatenate([res[0][:_N], res[1][:_N]], axis=1)
    y1 = jnp.concatenate([res[2][:_N], res[3][:_N]], axis=1)
    return (y0, y1) + tuple(res[4:])


def _dot(a, b):
    return jnp.dot(a, b, preferred_element_type=jnp.float32)


def _tc1_body(dp_ref, of_ref, *refs):
    (wa1, ba1, wa2, ba2, ga, gba,
     wl1, bl1, wl2, bl2, gl, gbl,
     wv1, bv1, wv2, bv2, gv, gbv,
     p1c, b1c, p2c, b2c, fc, bfc,
     p1a, b1a, p2a, b2a, fa, bfa,
     p1l, b1l, p2l, b2l, fl, bfl,
     p1v, b1v, p2v, b2v, fv, bfv,
     gwca, gwxa, gba2, gwcl, gwxl, gbl2, gwcv, gwxv, gbv2) = refs[:-1]
    out_ref = refs[-1]

    dp = dp_ref[...]
    of = of_ref[...]

    def enc(x, w1, b1, w2, b2, g, bb):
        h = jnp.maximum(_dot(x, w1[...]) + b1[...], 0.0)
        h = _dot(h, w2[...]) + b2[...]
        mu = jnp.mean(h, axis=-1, keepdims=True)
        var = jnp.mean((h - mu) * (h - mu), axis=-1, keepdims=True)
        return (h - mu) * lax.rsqrt(var + 1e-5) * g[...] + bb[...]

    ea = enc(dp[:, 0:128], wa1, ba1, wa2, ba2, ga, gba)
    el = enc(dp[:, 0:896], wl1, bl1, wl2, bl2, gl, gbl)
    ev = enc(dp[:, 768:1408], wv1, bv1, wv2, bv2, gv, gbv)

    def projfus(x, p1, b1, p2, b2, f, bf):
        j = _dot(jnp.maximum(_dot(x, p1[...]) + b1[...], 0.0), p2[...]) + b2[...]
        return _dot(j, f[...]) + bf[...]

    d_con = projfus(of, p1c, b1c, p2c, b2c, fc, bfc)
    d_a = projfus(ea, p1a, b1a, p2a, b2a, fa, bfa)
    d_l = projfus(el, p1l, b1l, p2l, b2l, fl, bfl)
    d_v = projfus(ev, p1v, b1v, p2v, b2v, fv, bfv)

    def gate(dx, gwc, gwx, gb):
        return (jnp.sum(d_con * gwc[...], axis=-1, keepdims=True)
                + jnp.sum(dx * gwx[...], axis=-1, keepdims=True) + gb[...])

    sa = gate(d_a, gwca, gwxa, gba2)
    sl = gate(d_l, gwcl, gwxl, gbl2)
    sv = gate(d_v, gwcv, gwxv, gbv2)
    m = jnp.maximum(jnp.maximum(sa, sl), sv)
    ea_ = jnp.exp(sa - m)
    el_ = jnp.exp(sl - m)
    ev_ = jnp.exp(sv - m)
    s = ea_ + el_ + ev_
    out_ref[...] = (ea_ * d_a + el_ * d_l + ev_ * d_v) / s


def _full_spec(a):
    nd = a.ndim
    return pl.BlockSpec(a.shape, lambda i, _nd=nd: (0,) * _nd)


def _row_spec(bn, d):
    return pl.BlockSpec((bn, d), lambda i: (i, 0))


def _tc1(dpad, of, wlist):
    in_specs = [_row_spec(_BN1, 1408), _row_spec(_BN1, _H)]
    in_specs += [_full_spec(w) for w in wlist]
    return pl.pallas_call(
        _tc1_body,
        grid=(_N // _BN1,),
        in_specs=in_specs,
        out_specs=_row_spec(_BN1, _H),
        out_shape=jax.ShapeDtypeStruct((_N, _H), jnp.float32),
    )(dpad, of, *wlist)


def _tc2_body(aj_ref, az_ref, dg_ref, wj, bj, wz, bz, oj_ref, oz_ref):
    d = jnp.maximum(dg_ref[:, 0:1], 1.0)
    aj = aj_ref[...] / d
    az = az_ref[...] / d
    oj_ref[...] = jnp.maximum(_dot(aj, wj[...]) + bj[...], 0.0)
    oz_ref[...] = jnp.maximum(_dot(az, wz[...]) + bz[...], 0.0)


def _tc2(aj, az, deg16, wj, bj, wz, bz):
    wl = [wj, bj, wz, bz]
    in_specs = [_row_spec(_BN2, _H), _row_spec(_BN2, _H), _row_spec(_BN2, 16)]
    in_specs += [_full_spec(w) for w in wl]
    return pl.pallas_call(
        _tc2_body,
        grid=(_N // _BN2,),
        in_specs=in_specs,
        out_specs=(_row_spec(_BN2, _H), _row_spec(_BN2, _H)),
        out_shape=(jax.ShapeDtypeStruct((_N, _H), jnp.float32),
                   jax.ShapeDtypeStruct((_N, _H), jnp.float32)),
    )(aj, az, deg16, *wl)


def _tc3_body(aj_ref, az_ref, dg_ref, wj, bj, wz, bz,
              p1a, p1b, b1, p2, b2, ow, ob, o_ref):
    d = jnp.maximum(dg_ref[:, 0:1], 1.0)
    hj = jnp.maximum(_dot(aj_ref[...] / d, wj[...]) + bj[...], 0.0)
    hz = jnp.maximum(_dot(az_ref[...] / d, wz[...]) + bz[...], 0.0)
    h = jnp.maximum(_dot(hj, p1a[...]) + _dot(hz, p1b[...]) + b1[...], 0.0)
    h = _dot(h, p2[...]) + b2[...]
    z = _dot(h, ow[...]) + ob[...]
    m = jnp.max(z, axis=-1, keepdims=True)
    lse = m + jnp.log(jnp.sum(jnp.exp(z - m), axis=-1, keepdims=True))
    o_ref[...] = z - lse


def _tc3(aj, az, deg16, wl):
    in_specs = [_row_spec(_BN2, _H), _row_spec(_BN2, _H), _row_spec(_BN2, 16)]
    in_specs += [_full_spec(w) for w in wl]
    return pl.pallas_call(
        _tc3_body,
        grid=(_N // _BN2,),
        in_specs=in_specs,
        out_specs=_row_spec(_BN2, _H),
        out_shape=jax.ShapeDtypeStruct((_N, _H), jnp.float32),
    )(aj, az, deg16, *wl)


def kernel(out_features, data_input, edge_index, params):
    f32 = jnp.float32
    src2 = edge_index[0].reshape(_NS, _CH, _B)
    dst2 = edge_index[1].reshape(_NS, _CH, _B)
    dpad = jnp.pad(data_input, ((0, 0), (0, 28)))

    def b2d(b):
        return b.reshape(1, -1)

    p = params
    wa1 = jnp.pad(p['enc_a']['l1'][0], ((0, 28), (0, 0)))
    wl1 = jnp.pad(p['enc_l']['l1'][0], ((100, 28), (0, 0)))
    wv1 = jnp.pad(p['enc_v']['l1'][0], ((100, 28), (0, 0)))

    def encw(name, w1):
        e = p[name]
        return [w1, b2d(e['l1'][1]), e['l2'][0], b2d(e['l2'][1]),
                b2d(e['ln_g']), b2d(e['ln_b'])]

    def pfw(tag):
        return [p['proj1_' + tag][0], b2d(p['proj1_' + tag][1]),
                p['proj2_' + tag][0], b2d(p['proj2_' + tag][1]),
                p['fus_' + tag][0], b2d(p['fus_' + tag][1])]

    def gatew(tag):
        w = p['w_' + tag][0]
        return [w[:_H].reshape(1, _H), w[_H:].reshape(1, _H),
                p['w_' + tag][1].reshape(1, 1)]

    wlist = (encw('enc_a', wa1) + encw('enc_l', wl1) + encw('enc_v', wv1)
             + pfw('con') + pfw('a') + pfw('l') + pfw('v')
             + gatew('a') + gatew('l') + gatew('v'))

    j_fused = _tc1(dpad, out_features, wlist)

    zfeat = jnp.zeros((_RPT, _HH), f32)
    zdeg = jnp.zeros((_RPT, 16), f32)
    ones80 = jnp.ones((_B, 16), f32)

    a1j, a1z, deg16 = _agg_pair(_make_agg(True), j_fused, out_features,
                                src2, dst2, (zfeat, ones80, zdeg))
    deg16 = deg16[:_N]

    h1j, h1z = _tc2(a1j, a1z, deg16,
                    p['gcn_j1'][0], b2d(p['gcn_j1'][1]),
                    p['gcn_z1'][0], b2d(p['gcn_z1'][1]))

    a2j, a2z = _agg_pair(_make_agg(False), h1j, h1z, src2, dst2, (zfeat,))

    ow = jnp.pad(p['out_layer'][0], ((0, 0), (0, _H - _C)))
    ob = jnp.concatenate([p['out_layer'][1],
                          jnp.full((_H - _C,), -1e30, f32)]).reshape(1, _H)
    p1 = p['proj1_out'][0]
    w3 = [p['gcn_j2'][0], b2d(p['gcn_j2'][1]),
          p['gcn_z2'][0], b2d(p['gcn_z2'][1]),
          p1[:_H], p1[_H:], b2d(p['proj1_out'][1]),
          p['proj2_out'][0], b2d(p['proj2_out'][1]),
          ow, ob]

    out128 = _tc3(a2j, a2z, deg16, w3)
    return out128[:, :_C]


# speedup vs baseline: 5.5922x; 1.5089x over previous
import jax
import jax.numpy as jnp
from jax import lax
from jax.experimental import pallas as pl
from jax.experimental.pallas import tpu as pltpu
from jax.experimental.pallas import tpu_sc as plsc

_N = 10000
_E = 320000
_H = 128
_C = 6

_NS = 16
_B = 80
_EPS = _E // _NS
_CH = _EPS // _B
_NP = 10240
_RPT = _NP // _NS
_HH = _H // 2

_BN1 = 1000
_BN2 = 2000


def _make_agg(with_deg):
    mesh = plsc.VectorSubcoreMesh(core_axis_name="c", subcore_axis_name="s",
                                  num_cores=2, num_subcores=_NS)
    outs = [jax.ShapeDtypeStruct((_NP, _HH), jnp.float32)
            for _ in range(4)]
    scratch = [
        pltpu.VMEM((_CH, _B), jnp.int32),
        pltpu.VMEM((_CH, _B), jnp.int32),
        pltpu.VMEM((_B, _HH), jnp.float32),
        pltpu.VMEM((_B, _HH), jnp.float32),
        pltpu.VMEM_SHARED((_NP, _HH), jnp.float32),
        pltpu.SemaphoreType.DMA,
        pltpu.SemaphoreType.DMA,
    ]
    if with_deg:
        outs.append(jax.ShapeDtypeStruct((_NP, 16), jnp.float32))
        scratch += [
            pltpu.VMEM((_B, 16), jnp.float32),
            pltpu.VMEM_SHARED((_NP, 16), jnp.float32),
        ]

    def _pass(sid, xref, yref, zfeat, sidx, didx, r0, r1, acc, s0, s1, degp):
        pltpu.sync_copy(zfeat, acc.at[pl.ds(sid * _RPT, _RPT)])
        if degp is not None:
            ones80, zdeg, degy, onesv, accd = degp
            pltpu.sync_copy(zdeg, accd.at[pl.ds(sid * _RPT, _RPT)])
            pltpu.sync_copy(ones80, onesv)
        plsc.subcore_barrier()

        pltpu.async_copy(xref.at[sidx.at[0]], r0, s0)

        def scat(i, buf):
            pltpu.sync_copy(buf, acc.at[didx.at[i]], add=True)
            if degp is not None:
                pltpu.sync_copy(degp[3], degp[4].at[didx.at[i]], add=True)

        def pair(k, carry):
            i0 = 2 * k
            d1 = pltpu.async_copy(xref.at[sidx.at[i0 + 1]], r1, s1)
            pltpu.make_async_copy(xref.at[sidx.at[i0]], r0, s0).wait()
            scat(i0, r0)
            nxt = jnp.minimum(i0 + 2, _CH - 1)
            pltpu.async_copy(xref.at[sidx.at[nxt]], r0, s0)
            d1.wait()
            scat(i0 + 1, r1)
            return carry

        lax.fori_loop(0, _CH // 2, pair, 0)
        pltpu.make_async_copy(xref.at[sidx.at[_CH - 1]], r0, s0).wait()
        plsc.subcore_barrier()
        pltpu.sync_copy(acc.at[pl.ds(sid * _RPT, _RPT)],
                        yref.at[pl.ds(sid * _RPT, _RPT)])
        if degp is not None:
            pltpu.sync_copy(degp[4].at[pl.ds(sid * _RPT, _RPT)],
                            degp[2].at[pl.ds(sid * _RPT, _RPT)])

    def _core(sid, xa, xb, ya, yb, zfeat, sidx, didx, r0, r1, acc,
              s0, s1, degp):
        _pass(sid, xa, ya, zfeat, sidx, didx, r0, r1, acc, s0, s1, degp)
        plsc.subcore_barrier()
        _pass(sid, xb, yb, zfeat, sidx, didx, r0, r1, acc, s0, s1, None)

    if with_deg:
        def body(x0a, x0b, x1a, x1b, src2, dst2, zfeat, ones80, zdeg,
                 y0a, y0b, y1a, y1b, degy,
                 sidx, didx, r0, r1, acc, s0, s1, onesv, accd):
            cid = lax.axis_index("c")
            sid = lax.axis_index("s")
            pltpu.sync_copy(src2.at[sid], sidx)
            pltpu.sync_copy(dst2.at[sid], didx)

            @pl.when(cid == 0)
            def _():
                _core(sid, x0a, x0b, y0a, y0b, zfeat, sidx, didx, r0, r1,
                      acc, s0, s1, (ones80, zdeg, degy, onesv, accd))

            @pl.when(cid == 1)
            def _():
                _core(sid, x1a, x1b, y1a, y1b, zfeat, sidx, didx, r0, r1,
                      acc, s0, s1, None)
    else:
        def body(x0a, x0b, x1a, x1b, src2, dst2, zfeat,
                 y0a, y0b, y1a, y1b,
                 sidx, didx, r0, r1, acc, s0, s1):
            cid = lax.axis_index("c")
            sid = lax.axis_index("s")
            pltpu.sync_copy(src2.at[sid], sidx)
            pltpu.sync_copy(dst2.at[sid], didx)

            @pl.when(cid == 0)
            def _():
                _core(sid, x0a, x0b, y0a, y0b, zfeat, sidx, didx, r0, r1,
                      acc, s0, s1, None)

            @pl.when(cid == 1)
            def _():
                _core(sid, x1a, x1b, y1a, y1b, zfeat, sidx, didx, r0, r1,
                      acc, s0, s1, None)

    return pl.kernel(body, out_type=tuple(outs), mesh=mesh,
                     scratch_types=tuple(scratch),
                     compiler_params=pltpu.CompilerParams(
                         use_tc_tiling_on_sc=False))


def _dot(a, b):
    return jnp.dot(a, b, preferred_element_type=jnp.float32)


def _tc1_body(dp_ref, of_ref, *refs):
    (wa1, ba1, wa2, ba2, ga, gba,
     wl1, bl1, wl2, bl2, gl, gbl,
     wv1, bv1, wv2, bv2, gv, gbv,
     p1c, b1c, p2c, b2c, fc, bfc,
     p1a, b1a, p2a, b2a, fa, bfa,
     p1l, b1l, p2l, b2l, fl, bfl,
     p1v, b1v, p2v, b2v, fv, bfv,
     gwca, gwxa, gba2, gwcl, gwxl, gbl2, gwcv, gwxv, gbv2) = refs[:-4]
    ja_ref, jb_ref, ofa_ref, ofb_ref = refs[-4:]

    dp = dp_ref[...]
    of = of_ref[...]
    ofa_ref[...] = of[:, :_HH]
    ofb_ref[...] = of[:, _HH:]

    def enc(x, w1, b1, w2, b2, g, bb):
        h = jnp.maximum(_dot(x, w1[...]) + b1[...], 0.0)
        h = _dot(h, w2[...]) + b2[...]
        mu = jnp.mean(h, axis=-1, keepdims=True)
        var = jnp.mean((h - mu) * (h - mu), axis=-1, keepdims=True)
        return (h - mu) * lax.rsqrt(var + 1e-5) * g[...] + bb[...]

    ea = enc(dp[:, 0:128], wa1, ba1, wa2, ba2, ga, gba)
    el = enc(dp[:, 0:896], wl1, bl1, wl2, bl2, gl, gbl)
    ev = enc(dp[:, 768:1408], wv1, bv1, wv2, bv2, gv, gbv)

    def projfus(x, p1, b1, p2, b2, f, bf):
        j = _dot(jnp.maximum(_dot(x, p1[...]) + b1[...], 0.0), p2[...]) + b2[...]
        return _dot(j, f[...]) + bf[...]

    d_con = projfus(of, p1c, b1c, p2c, b2c, fc, bfc)
    d_a = projfus(ea, p1a, b1a, p2a, b2a, fa, bfa)
    d_l = projfus(el, p1l, b1l, p2l, b2l, fl, bfl)
    d_v = projfus(ev, p1v, b1v, p2v, b2v, fv, bfv)

    def gate(dx, gwc, gwx, gb):
        return (jnp.sum(d_con * gwc[...], axis=-1, keepdims=True)
                + jnp.sum(dx * gwx[...], axis=-1, keepdims=True) + gb[...])

    sa = gate(d_a, gwca, gwxa, gba2)
    sl = gate(d_l, gwcl, gwxl, gbl2)
    sv = gate(d_v, gwcv, gwxv, gbv2)
    m = jnp.maximum(jnp.maximum(sa, sl), sv)
    ea_ = jnp.exp(sa - m)
    el_ = jnp.exp(sl - m)
    ev_ = jnp.exp(sv - m)
    s = ea_ + el_ + ev_
    jf = (ea_ * d_a + el_ * d_l + ev_ * d_v) / s
    ja_ref[...] = jf[:, :_HH]
    jb_ref[...] = jf[:, _HH:]


def _full_spec(a):
    nd = a.ndim
    return pl.BlockSpec(a.shape, lambda i, _nd=nd: (0,) * _nd)


def _row_spec(bn, d):
    return pl.BlockSpec((bn, d), lambda i: (i, 0))


def _half_struct():
    return jax.ShapeDtypeStruct((_N, _HH), jnp.float32)


def _tc1(dpad, of, wlist):
    in_specs = [_row_spec(_BN1, 1408), _row_spec(_BN1, _H)]
    in_specs += [_full_spec(w) for w in wlist]
    return pl.pallas_call(
        _tc1_body,
        grid=(_N // _BN1,),
        in_specs=in_specs,
        out_specs=tuple(_row_spec(_BN1, _HH) for _ in range(4)),
        out_shape=tuple(_half_struct() for _ in range(4)),
    )(dpad, of, *wlist)


def _tc2_body(aja_ref, ajb_ref, aza_ref, azb_ref, dg_ref,
              wja, wjb, bj, wza, wzb, bz,
              oja_ref, ojb_ref, oza_ref, ozb_ref):
    d = jnp.maximum(dg_ref[:, 0:1], 1.0)
    r = 1.0 / d
    hj = jnp.maximum(_dot(aja_ref[...] * r, wja[...])
                     + _dot(ajb_ref[...] * r, wjb[...]) + bj[...], 0.0)
    hz = jnp.maximum(_dot(aza_ref[...] * r, wza[...])
                     + _dot(azb_ref[...] * r, wzb[...]) + bz[...], 0.0)
    oja_ref[...] = hj[:, :_HH]
    ojb_ref[...] = hj[:, _HH:]
    oza_ref[...] = hz[:, :_HH]
    ozb_ref[...] = hz[:, _HH:]


def _tc2(aggs, deg16, wj, bj, wz, bz):
    wl = [wj[:_HH], wj[_HH:], bj, wz[:_HH], wz[_HH:], bz]
    in_specs = [_row_spec(_BN2, _HH)] * 4 + [_row_spec(_BN2, 16)]
    in_specs += [_full_spec(w) for w in wl]
    return pl.pallas_call(
        _tc2_body,
        grid=(_N // _BN2,),
        in_specs=in_specs,
        out_specs=tuple(_row_spec(_BN2, _HH) for _ in range(4)),
        out_shape=tuple(_half_struct() for _ in range(4)),
    )(*aggs, deg16, *wl)


def _tc3_body(aja_ref, ajb_ref, aza_ref, azb_ref, dg_ref,
              wja, wjb, bj, wza, wzb, bz,
              p1a, p1b, b1, p2, b2, ow, ob, o_ref):
    d = jnp.maximum(dg_ref[:, 0:1], 1.0)
    r = 1.0 / d
    hj = jnp.maximum(_dot(aja_ref[...] * r, wja[...])
                     + _dot(ajb_ref[...] * r, wjb[...]) + bj[...], 0.0)
    hz = jnp.maximum(_dot(aza_ref[...] * r, wza[...])
                     + _dot(azb_ref[...] * r, wzb[...]) + bz[...], 0.0)
    h = jnp.maximum(_dot(hj, p1a[...]) + _dot(hz, p1b[...]) + b1[...], 0.0)
    h = _dot(h, p2[...]) + b2[...]
    z = _dot(h, ow[...]) + ob[...]
    m = jnp.max(z, axis=-1, keepdims=True)
    lse = m + jnp.log(jnp.sum(jnp.exp(z - m), axis=-1, keepdims=True))
    o_ref[...] = z - lse


def _tc3(aggs, deg16, wl):
    in_specs = [_row_spec(_BN2, _HH)] * 4 + [_row_spec(_BN2, 16)]
    in_specs += [_full_spec(w) for w in wl]
    return pl.pallas_call(
        _tc3_body,
        grid=(_N // _BN2,),
        in_specs=in_specs,
        out_specs=_row_spec(_BN2, _H),
        out_shape=jax.ShapeDtypeStruct((_N, _H), jnp.float32),
    )(*aggs, deg16, *wl)


def kernel(out_features, data_input, edge_index, params):
    f32 = jnp.float32
    src2 = edge_index[0].reshape(_NS, _CH, _B)
    dst2 = edge_index[1].reshape(_NS, _CH, _B)
    dpad = jnp.pad(data_input, ((0, 0), (0, 28)))

    def b2d(b):
        return b.reshape(1, -1)

    p = params
    wa1 = jnp.pad(p['enc_a']['l1'][0], ((0, 28), (0, 0)))
    wl1 = jnp.pad(p['enc_l']['l1'][0], ((100, 28), (0, 0)))
    wv1 = jnp.pad(p['enc_v']['l1'][0], ((100, 28), (0, 0)))

    def encw(name, w1):
        e = p[name]
        return [w1, b2d(e['l1'][1]), e['l2'][0], b2d(e['l2'][1]),
                b2d(e['ln_g']), b2d(e['ln_b'])]

    def pfw(tag):
        return [p['proj1_' + tag][0], b2d(p['proj1_' + tag][1]),
                p['proj2_' + tag][0], b2d(p['proj2_' + tag][1]),
                p['fus_' + tag][0], b2d(p['fus_' + tag][1])]

    def gatew(tag):
        w = p['w_' + tag][0]
        return [w[:_H].reshape(1, _H), w[_H:].reshape(1, _H),
                p['w_' + tag][1].reshape(1, 1)]

    wlist = (encw('enc_a', wa1) + encw('enc_l', wl1) + encw('enc_v', wv1)
             + pfw('con') + pfw('a') + pfw('l') + pfw('v')
             + gatew('a') + gatew('l') + gatew('v'))

    ja, jb, ofa, ofb = _tc1(dpad, out_features, wlist)

    zfeat = jnp.zeros((_RPT, _HH), f32)
    zdeg = jnp.zeros((_RPT, 16), f32)
    ones80 = jnp.ones((_B, 16), f32)

    a1 = _make_agg(True)(ja, jb, ofa, ofb, src2, dst2, zfeat, ones80, zdeg)
    deg16 = a1[4]

    h1 = _tc2(a1[:4], deg16,
              p['gcn_j1'][0], b2d(p['gcn_j1'][1]),
              p['gcn_z1'][0], b2d(p['gcn_z1'][1]))

    a2 = _make_agg(False)(*h1, src2, dst2, zfeat)

    ow = jnp.pad(p['out_layer'][0], ((0, 0), (0, _H - _C)))
    ob = jnp.concatenate([p['out_layer'][1],
                          jnp.full((_H - _C,), -1e30, f32)]).reshape(1, _H)
    p1 = p['proj1_out'][0]
    w3 = [p['gcn_j2'][0][:_HH], p['gcn_j2'][0][_HH:], b2d(p['gcn_j2'][1]),
          p['gcn_z2'][0][:_HH], p['gcn_z2'][0][_HH:], b2d(p['gcn_z2'][1]),
          p1[:_H], p1[_H:], b2d(p['proj1_out'][1]),
          p['proj2_out'][0], b2d(p['proj2_out'][1]),
          ow, ob]

    out128 = _tc3(a2, deg16, w3)
    return out128[:, :_C]


# speedup vs baseline: 8.4500x; 1.5110x over previous
import jax
import jax.numpy as jnp
from jax import lax
from jax.experimental import pallas as pl
from jax.experimental.pallas import tpu as pltpu
from jax.experimental.pallas import tpu_sc as plsc

_N = 10000
_E = 320000
_H = 128
_C = 6

_NS = 16
_B = 125
_EPS = _E // _NS
_CH = _EPS // _B
_NB = 2
_NP = 10240
_RPT = _NP // _NS
_HH = _H // 2

_BN0 = 1024
_BN1 = 1000
_BN2 = 2000


def _make_agg(with_deg):
    mesh = plsc.VectorSubcoreMesh(core_axis_name="c", subcore_axis_name="s",
                                  num_cores=2, num_subcores=_NS)
    outs = [jax.ShapeDtypeStruct((_NP, _HH), jnp.float32)
            for _ in range(4)]
    scratch = [
        pltpu.VMEM((_CH, _B), jnp.int32),
        pltpu.VMEM((_CH, _B), jnp.int32),
        pltpu.VMEM_SHARED((_NP, _HH), jnp.float32),
    ]
    scratch += [pltpu.VMEM((_B, _HH), jnp.float32) for _ in range(2 * _NB)]
    scratch += [pltpu.SemaphoreType.DMA for _ in range(5)]
    if with_deg:
        outs.append(jax.ShapeDtypeStruct((_NP, 16), jnp.float32))
        scratch += [
            pltpu.VMEM((_B, 16), jnp.float32),
            pltpu.VMEM_SHARED((_NP, 16), jnp.float32),
        ]

    def _pass(sid, xref, yref, zfeat, sidx, didx, acc, rings, sems, degp):
        ra, rb = rings
        gsa, gsb, ssa, ssb, sd = sems
        pltpu.sync_copy(zfeat, acc.at[pl.ds(sid * _RPT, _RPT)])
        if degp is not None:
            ones_h, zdeg, degy, onesv, accd = degp
            pltpu.sync_copy(zdeg, accd.at[pl.ds(sid * _RPT, _RPT)])
            pltpu.sync_copy(ones_h, onesv)
        plsc.subcore_barrier()

        def gather(i, buf, sem):
            pltpu.async_copy(xref.at[sidx.at[i]], buf, sem)

        def gwait(buf, sem):
            pltpu.make_async_copy(xref.at[sidx.at[0]], buf, sem).wait()

        for b in range(_NB):
            gather(b, ra[b], gsa)

        def superstep(base, rs, rg, gs_s, gs_g, ss_s):
            for b in range(_NB):
                gwait(rs[b], gs_s)
            for b in range(_NB):
                pltpu.async_copy(rs[b], acc.at[didx.at[base + b]], ss_s,
                                 add=True)
            if degp is not None:
                for b in range(_NB):
                    pltpu.async_copy(degp[3], degp[4].at[didx.at[base + b]],
                                     sd, add=True)
            for b in range(_NB):
                nxt = jnp.minimum(base + _NB + b, _CH - 1)
                gather(nxt, rg[b], gs_g)
            for b in range(_NB):
                pltpu.make_async_copy(rs[b], acc.at[didx.at[base + b]],
                                      ss_s).wait()
            if degp is not None:
                for b in range(_NB):
                    pltpu.make_async_copy(degp[3],
                                          degp[4].at[didx.at[base + b]],
                                          sd).wait()

        def two_steps(k, carry):
            superstep(2 * _NB * k, ra, rb, gsa, gsb, ssa)
            superstep(2 * _NB * k + _NB, rb, ra, gsb, gsa, ssb)
            return carry

        lax.fori_loop(0, _CH // (2 * _NB), two_steps, 0)
        for b in range(_NB):
            gwait(ra[b], gsa)
        plsc.subcore_barrier()
        pltpu.sync_copy(acc.at[pl.ds(sid * _RPT, _RPT)],
                        yref.at[pl.ds(sid * _RPT, _RPT)])
        if degp is not None:
            pltpu.sync_copy(degp[4].at[pl.ds(sid * _RPT, _RPT)],
                            degp[2].at[pl.ds(sid * _RPT, _RPT)])

    def _core(sid, xa, xb, ya, yb, zfeat, sidx, didx, acc, rings, sems,
              degp):
        _pass(sid, xa, ya, zfeat, sidx, didx, acc, rings, sems, degp)
        plsc.subcore_barrier()
        _pass(sid, xb, yb, zfeat, sidx, didx, acc, rings, sems, None)

    def _body_common(cid, sid, x0a, x0b, x1a, x1b, src2, dst2, zfeat,
                     y0a, y0b, y1a, y1b, sidx, didx, acc, rings, sems,
                     degp):
        pltpu.sync_copy(src2.at[sid], sidx)
        pltpu.sync_copy(dst2.at[sid], didx)

        @pl.when(cid == 0)
        def _():
            _core(sid, x0a, x0b, y0a, y0b, zfeat, sidx, didx, acc, rings,
                  sems, degp)

        @pl.when(cid == 1)
        def _():
            _core(sid, x1a, x1b, y1a, y1b, zfeat, sidx, didx, acc, rings,
                  sems, None)

    if with_deg:
        def body(x0a, x0b, x1a, x1b, src2, dst2, zfeat, ones_h, zdeg,
                 y0a, y0b, y1a, y1b, degy,
                 sidx, didx, acc, *rest):
            bufs = rest[:2 * _NB]
            sems = rest[2 * _NB:2 * _NB + 5]
            onesv, accd = rest[2 * _NB + 5:]
            cid = lax.axis_index("c")
            sid = lax.axis_index("s")
            _body_common(cid, sid, x0a, x0b, x1a, x1b, src2, dst2, zfeat,
                         y0a, y0b, y1a, y1b, sidx, didx, acc,
                         (bufs[:_NB], bufs[_NB:]), sems,
                         (ones_h, zdeg, degy, onesv, accd))
    else:
        def body(x0a, x0b, x1a, x1b, src2, dst2, zfeat,
                 y0a, y0b, y1a, y1b,
                 sidx, didx, acc, *rest):
            bufs = rest[:2 * _NB]
            sems = rest[2 * _NB:2 * _NB + 5]
            cid = lax.axis_index("c")
            sid = lax.axis_index("s")
            _body_common(cid, sid, x0a, x0b, x1a, x1b, src2, dst2, zfeat,
                         y0a, y0b, y1a, y1b, sidx, didx, acc,
                         (bufs[:_NB], bufs[_NB:]), sems, None)

    return pl.kernel(body, out_type=tuple(outs), mesh=mesh,
                     scratch_types=tuple(scratch),
                     compiler_params=pltpu.CompilerParams(
                         use_tc_tiling_on_sc=False))


def _dot(a, b):
    return jnp.dot(a, b, preferred_element_type=jnp.float32)


def _tc0_body(dt_ref, w_ref, b_ref, o_ref):
    t = dt_ref[...]
    h = lax.dot_general(t, w_ref[...], (((0,), (0,)), ((), ())),
                        preferred_element_type=jnp.float32)
    o_ref[...] = jnp.maximum(h + b_ref[...], 0.0)


def _tc0(dT, wbig, bbig):
    grid = (_N + _BN0 - 1) // _BN0
    return pl.pallas_call(
        _tc0_body,
        grid=(grid,),
        in_specs=[pl.BlockSpec((1380, _BN0), lambda i: (0, i)),
                  pl.BlockSpec((1380, 768), lambda i: (0, 0)),
                  pl.BlockSpec((1, 768), lambda i: (0, 0))],
        out_specs=pl.BlockSpec((_BN0, 768), lambda i: (i, 0)),
        out_shape=jax.ShapeDtypeStruct((_N, 768), jnp.float32),
    )(dT, wbig, bbig)


def _tc1_body(h1_ref, of_ref, *refs):
    (wa2, ba2, ga, gba,
     wl2, bl2, gl, gbl,
     wv2, bv2, gv, gbv,
     p1c, b1c, p2c, b2c, fc, bfc,
     p1a, b1a, p2a, b2a, fa, bfa,
     p1l, b1l, p2l, b2l, fl, bfl,
     p1v, b1v, p2v, b2v, fv, bfv,
     gwca, gwxa, gba2, gwcl, gwxl, gbl2, gwcv, gwxv, gbv2) = refs[:-4]
    ja_ref, jb_ref, ofa_ref, ofb_ref = refs[-4:]

    h1 = h1_ref[...]
    of = of_ref[...]
    ofa_ref[...] = of[:, :_HH]
    ofb_ref[...] = of[:, _HH:]

    def enc(h, w2, b2, g, bb):
        h = _dot(h, w2[...]) + b2[...]
        mu = jnp.mean(h, axis=-1, keepdims=True)
        var = jnp.mean((h - mu) * (h - mu), axis=-1, keepdims=True)
        return (h - mu) * lax.rsqrt(var + 1e-5) * g[...] + bb[...]

    ea = enc(h1[:, 0:256], wa2, ba2, ga, gba)
    el = enc(h1[:, 256:512], wl2, bl2, gl, gbl)
    ev = enc(h1[:, 512:768], wv2, bv2, gv, gbv)

    def projfus(x, p1, b1, p2, b2, f, bf):
        j = _dot(jnp.maximum(_dot(x, p1[...]) + b1[...], 0.0), p2[...]) + b2[...]
        return _dot(j, f[...]) + bf[...]

    d_con = projfus(of, p1c, b1c, p2c, b2c, fc, bfc)
    d_a = projfus(ea, p1a, b1a, p2a, b2a, fa, bfa)
    d_l = projfus(el, p1l, b1l, p2l, b2l, fl, bfl)
    d_v = projfus(ev, p1v, b1v, p2v, b2v, fv, bfv)

    def gate(dx, gwc, gwx, gb):
        return (jnp.sum(d_con * gwc[...], axis=-1, keepdims=True)
                + jnp.sum(dx * gwx[...], axis=-1, keepdims=True) + gb[...])

    sa = gate(d_a, gwca, gwxa, gba2)
    sl = gate(d_l, gwcl, gwxl, gbl2)
    sv = gate(d_v, gwcv, gwxv, gbv2)
    m = jnp.maximum(jnp.maximum(sa, sl), sv)
    ea_ = jnp.exp(sa - m)
    el_ = jnp.exp(sl - m)
    ev_ = jnp.exp(sv - m)
    s = ea_ + el_ + ev_
    jf = (ea_ * d_a + el_ * d_l + ev_ * d_v) / s
    ja_ref[...] = jf[:, :_HH]
    jb_ref[...] = jf[:, _HH:]


def _full_spec(a):
    nd = a.ndim
    return pl.BlockSpec(a.shape, lambda i, _nd=nd: (0,) * _nd)


def _row_spec(bn, d):
    return pl.BlockSpec((bn, d), lambda i: (i, 0))


def _half_struct():
    return jax.ShapeDtypeStruct((_N, _HH), jnp.float32)


def _tc1(h_all, of, wlist):
    in_specs = [_row_spec(_BN1, 768), _row_spec(_BN1, _H)]
    in_specs += [_full_spec(w) for w in wlist]
    return pl.pallas_call(
        _tc1_body,
        grid=(_N // _BN1,),
        in_specs=in_specs,
        out_specs=tuple(_row_spec(_BN1, _HH) for _ in range(4)),
        out_shape=tuple(_half_struct() for _ in range(4)),
    )(h_all, of, *wlist)


def _tc2_body(aja_ref, ajb_ref, aza_ref, azb_ref, dg_ref,
              wja, wjb, bj, wza, wzb, bz,
              oja_ref, ojb_ref, oza_ref, ozb_ref):
    d = jnp.maximum(dg_ref[:, 0:1], 1.0)
    r = 1.0 / d
    hj = jnp.maximum(_dot(aja_ref[...] * r, wja[...])
                     + _dot(ajb_ref[...] * r, wjb[...]) + bj[...], 0.0)
    hz = jnp.maximum(_dot(aza_ref[...] * r, wza[...])
                     + _dot(azb_ref[...] * r, wzb[...]) + bz[...], 0.0)
    oja_ref[...] = hj[:, :_HH]
    ojb_ref[...] = hj[:, _HH:]
    oza_ref[...] = hz[:, :_HH]
    ozb_ref[...] = hz[:, _HH:]


def _tc2(aggs, deg16, wj, bj, wz, bz):
    wl = [wj[:_HH], wj[_HH:], bj, wz[:_HH], wz[_HH:], bz]
    in_specs = [_row_spec(_BN2, _HH)] * 4 + [_row_spec(_BN2, 16)]
    in_specs += [_full_spec(w) for w in wl]
    return pl.pallas_call(
        _tc2_body,
        grid=(_N // _BN2,),
        in_specs=in_specs,
        out_specs=tuple(_row_spec(_BN2, _HH) for _ in range(4)),
        out_shape=tuple(_half_struct() for _ in range(4)),
    )(*aggs, deg16, *wl)


def _tc3_body(aja_ref, ajb_ref, aza_ref, azb_ref, dg_ref,
              wja, wjb, bj, wza, wzb, bz,
              p1a, p1b, b1, p2, b2, ow, ob, o_ref):
    d = jnp.maximum(dg_ref[:, 0:1], 1.0)
    r = 1.0 / d
    hj = jnp.maximum(_dot(aja_ref[...] * r, wja[...])
                     + _dot(ajb_ref[...] * r, wjb[...]) + bj[...], 0.0)
    hz = jnp.maximum(_dot(aza_ref[...] * r, wza[...])
                     + _dot(azb_ref[...] * r, wzb[...]) + bz[...], 0.0)
    h = jnp.maximum(_dot(hj, p1a[...]) + _dot(hz, p1b[...]) + b1[...], 0.0)
    h = _dot(h, p2[...]) + b2[...]
    z = _dot(h, ow[...]) + ob[...]
    m = jnp.max(z, axis=-1, keepdims=True)
    lse = m + jnp.log(jnp.sum(jnp.exp(z - m), axis=-1, keepdims=True))
    o_ref[...] = z - lse


def _tc3(aggs, deg16, wl):
    in_specs = [_row_spec(_BN2, _HH)] * 4 + [_row_spec(_BN2, 16)]
    in_specs += [_full_spec(w) for w in wl]
    return pl.pallas_call(
        _tc3_body,
        grid=(_N // _BN2,),
        in_specs=in_specs,
        out_specs=_row_spec(_BN2, _H),
        out_shape=jax.ShapeDtypeStruct((_N, _H), jnp.float32),
    )(*aggs, deg16, *wl)


def kernel(out_features, data_input, edge_index, params):
    f32 = jnp.float32
    src2 = edge_index[0].reshape(_NS, _CH, _B)
    dst2 = edge_index[1].reshape(_NS, _CH, _B)

    def b2d(b):
        return b.reshape(1, -1)

    p = params
    wbig = jnp.zeros((1380, 768), f32)
    wbig = wbig.at[0:100, 0:256].set(p['enc_a']['l1'][0])
    wbig = wbig.at[100:868, 256:512].set(p['enc_l']['l1'][0])
    wbig = wbig.at[868:1380, 512:768].set(p['enc_v']['l1'][0])
    bbig = jnp.concatenate([p['enc_a']['l1'][1], p['enc_l']['l1'][1],
                            p['enc_v']['l1'][1]]).reshape(1, 768)

    h_all = _tc0(data_input.T, wbig, bbig)

    def encw(name):
        e = p[name]
        return [e['l2'][0], b2d(e['l2'][1]), b2d(e['ln_g']), b2d(e['ln_b'])]

    def pfw(tag):
        return [p['proj1_' + tag][0], b2d(p['proj1_' + tag][1]),
                p['proj2_' + tag][0], b2d(p['proj2_' + tag][1]),
                p['fus_' + tag][0], b2d(p['fus_' + tag][1])]

    def gatew(tag):
        w = p['w_' + tag][0]
        return [w[:_H].reshape(1, _H), w[_H:].reshape(1, _H),
                p['w_' + tag][1].reshape(1, 1)]

    wlist = (encw('enc_a') + encw('enc_l') + encw('enc_v')
             + pfw('con') + pfw('a') + pfw('l') + pfw('v')
             + gatew('a') + gatew('l') + gatew('v'))

    ja, jb, ofa, ofb = _tc1(h_all, out_features, wlist)

    zfeat = jnp.zeros((_RPT, _HH), f32)
    zdeg = jnp.zeros((_RPT, 16), f32)
    ones_h = jnp.ones((_B, 16), f32)

    a1 = _make_agg(True)(ja, jb, ofa, ofb, src2, dst2, zfeat, ones_h, zdeg)
    deg16 = a1[4]

    h1 = _tc2(a1[:4], deg16,
              p['gcn_j1'][0], b2d(p['gcn_j1'][1]),
              p['gcn_z1'][0], b2d(p['gcn_z1'][1]))

    a2 = _make_agg(False)(*h1, src2, dst2, zfeat)

    ow = jnp.pad(p['out_layer'][0], ((0, 0), (0, _H - _C)))
    ob = jnp.concatenate([p['out_layer'][1],
                          jnp.full((_H - _C,), -1e30, f32)]).reshape(1, _H)
    p1 = p['proj1_out'][0]
    w3 = [p['gcn_j2'][0][:_HH], p['gcn_j2'][0][_HH:], b2d(p['gcn_j2'][1]),
          p['gcn_z2'][0][:_HH], p['gcn_z2'][0][_HH:], b2d(p['gcn_z2'][1]),
          p1[:_H], p1[_H:], b2d(p['proj1_out'][1]),
          p['proj2_out'][0], b2d(p['proj2_out'][1]),
          ow, ob]

    out128 = _tc3(a2, deg16, w3)
    return out128[:, :_C]


# speedup vs baseline: 8.7361x; 1.0339x over previous
import jax
import jax.numpy as jnp
from jax import lax
from jax.experimental import pallas as pl
from jax.experimental.pallas import tpu as pltpu
from jax.experimental.pallas import tpu_sc as plsc

_N = 10000
_E = 320000
_H = 128
_C = 6

_NS = 16
_B = 125
_EPS = _E // _NS
_CH = _EPS // _B
_NB = 2
_NP = 10240
_RPT = _NP // _NS
_HH = _H // 2

_BN0 = 1024
_BN1 = 1000
_BN2 = 2000


def _make_agg(with_deg):
    mesh = plsc.VectorSubcoreMesh(core_axis_name="c", subcore_axis_name="s",
                                  num_cores=2, num_subcores=_NS)
    outs = [jax.ShapeDtypeStruct((_NP, _HH), jnp.float32)
            for _ in range(4)]
    scratch = [
        pltpu.VMEM((_CH, _B), jnp.int32),
        pltpu.VMEM((_CH, _B), jnp.int32),
        pltpu.VMEM_SHARED((_NP, _HH), jnp.float32),
    ]
    scratch += [pltpu.VMEM((_B, _HH), jnp.float32) for _ in range(2 * _NB)]
    scratch += [pltpu.SemaphoreType.DMA for _ in range(5)]
    if with_deg:
        outs.append(jax.ShapeDtypeStruct((_NP, 16), jnp.float32))
        scratch += [
            pltpu.VMEM((_B, 16), jnp.float32),
            pltpu.VMEM_SHARED((_NP, 16), jnp.float32),
        ]

    def _pass(sid, xref, yref, zfeat, sidx, didx, acc, rings, sems, degp):
        ra, rb = rings
        gsa, gsb, ssa, ssb, sd = sems
        pltpu.sync_copy(zfeat, acc.at[pl.ds(sid * _RPT, _RPT)])
        if degp is not None:
            ones_h, zdeg, degy, onesv, accd = degp
            pltpu.sync_copy(zdeg, accd.at[pl.ds(sid * _RPT, _RPT)])
            pltpu.sync_copy(ones_h, onesv)
        plsc.subcore_barrier()

        def gather(i, buf, sem):
            pltpu.async_copy(xref.at[sidx.at[i]], buf, sem)

        def gwait(buf, sem):
            pltpu.make_async_copy(xref.at[sidx.at[0]], buf, sem).wait()

        for b in range(_NB):
            gather(b, ra[b], gsa)

        def superstep(base, rs, rg, gs_s, gs_g, ss_s):
            for b in range(_NB):
                gwait(rs[b], gs_s)
            for b in range(_NB):
                pltpu.async_copy(rs[b], acc.at[didx.at[base + b]], ss_s,
                                 add=True)
            if degp is not None:
                for b in range(_NB):
                    pltpu.async_copy(degp[3], degp[4].at[didx.at[base + b]],
                                     sd, add=True)
            for b in range(_NB):
                nxt = jnp.minimum(base + _NB + b, _CH - 1)
                gather(nxt, rg[b], gs_g)
            for b in range(_NB):
                pltpu.make_async_copy(rs[b], acc.at[didx.at[base + b]],
                                      ss_s).wait()
            if degp is not None:
                for b in range(_NB):
                    pltpu.make_async_copy(degp[3],
                                          degp[4].at[didx.at[base + b]],
                                          sd).wait()

        def two_steps(k, carry):
            superstep(2 * _NB * k, ra, rb, gsa, gsb, ssa)
            superstep(2 * _NB * k + _NB, rb, ra, gsb, gsa, ssb)
            return carry

        lax.fori_loop(0, _CH // (2 * _NB), two_steps, 0)
        for b in range(_NB):
            gwait(ra[b], gsa)
        plsc.subcore_barrier()
        pltpu.sync_copy(acc.at[pl.ds(sid * _RPT, _RPT)],
                        yref.at[pl.ds(sid * _RPT, _RPT)])
        if degp is not None:
            pltpu.sync_copy(degp[4].at[pl.ds(sid * _RPT, _RPT)],
                            degp[2].at[pl.ds(sid * _RPT, _RPT)])

    def _core(sid, xa, xb, ya, yb, zfeat, sidx, didx, acc, rings, sems,
              degp):
        _pass(sid, xa, ya, zfeat, sidx, didx, acc, rings, sems, degp)
        plsc.subcore_barrier()
        _pass(sid, xb, yb, zfeat, sidx, didx, acc, rings, sems, None)

    def _body_common(cid, sid, x0a, x0b, x1a, x1b, src2, dst2, zfeat,
                     y0a, y0b, y1a, y1b, sidx, didx, acc, rings, sems,
                     degp):
        pltpu.sync_copy(src2.at[sid], sidx)
        pltpu.sync_copy(dst2.at[sid], didx)

        @pl.when(cid == 0)
        def _():
            _core(sid, x0a, x0b, y0a, y0b, zfeat, sidx, didx, acc, rings,
                  sems, degp)

        @pl.when(cid == 1)
        def _():
            _core(sid, x1a, x1b, y1a, y1b, zfeat, sidx, didx, acc, rings,
                  sems, None)

    if with_deg:
        def body(x0a, x0b, x1a, x1b, src2, dst2, zfeat, ones_h, zdeg,
                 y0a, y0b, y1a, y1b, degy,
                 sidx, didx, acc, *rest):
            bufs = rest[:2 * _NB]
            sems = rest[2 * _NB:2 * _NB + 5]
            onesv, accd = rest[2 * _NB + 5:]
            cid = lax.axis_index("c")
            sid = lax.axis_index("s")
            _body_common(cid, sid, x0a, x0b, x1a, x1b, src2, dst2, zfeat,
                         y0a, y0b, y1a, y1b, sidx, didx, acc,
                         (bufs[:_NB], bufs[_NB:]), sems,
                         (ones_h, zdeg, degy, onesv, accd))
    else:
        def body(x0a, x0b, x1a, x1b, src2, dst2, zfeat,
                 y0a, y0b, y1a, y1b,
                 sidx, didx, acc, *rest):
            bufs = rest[:2 * _NB]
            sems = rest[2 * _NB:2 * _NB + 5]
            cid = lax.axis_index("c")
            sid = lax.axis_index("s")
            _body_common(cid, sid, x0a, x0b, x1a, x1b, src2, dst2, zfeat,
                         y0a, y0b, y1a, y1b, sidx, didx, acc,
                         (bufs[:_NB], bufs[_NB:]), sems, None)

    return pl.kernel(body, out_type=tuple(outs), mesh=mesh,
                     scratch_types=tuple(scratch),
                     compiler_params=pltpu.CompilerParams(
                         use_tc_tiling_on_sc=False))


def _dot(a, b):
    return jnp.dot(a, b, preferred_element_type=jnp.float32)


def _tc0_body(dt_ref, wa_ref, wl_ref, wv_ref, b_ref, o_ref):
    t = dt_ref[...]

    def blk(lo, hi, w):
        return lax.dot_general(t[lo:hi], w[...], (((0,), (0,)), ((), ())),
                               preferred_element_type=jnp.float32)

    h = jnp.concatenate([blk(0, 100, wa_ref), blk(100, 868, wl_ref),
                         blk(868, 1380, wv_ref)], axis=1)
    o_ref[...] = jnp.maximum(h + b_ref[...], 0.0)


def _tc0(dT, wa, wl, wv, bbig):
    grid = (_N + _BN0 - 1) // _BN0
    return pl.pallas_call(
        _tc0_body,
        grid=(grid,),
        in_specs=[pl.BlockSpec((1380, _BN0), lambda i: (0, i)),
                  pl.BlockSpec((100, 256), lambda i: (0, 0)),
                  pl.BlockSpec((768, 256), lambda i: (0, 0)),
                  pl.BlockSpec((512, 256), lambda i: (0, 0)),
                  pl.BlockSpec((1, 768), lambda i: (0, 0))],
        out_specs=pl.BlockSpec((_BN0, 768), lambda i: (i, 0)),
        out_shape=jax.ShapeDtypeStruct((_N, 768), jnp.float32),
    )(dT, wa, wl, wv, bbig)


def _tc1_body(h1_ref, of_ref, *refs):
    (wa2, ba2, ga, gba,
     wl2, bl2, gl, gbl,
     wv2, bv2, gv, gbv,
     p1c, b1c, p2c, b2c, fc, bfc,
     p1a, b1a, p2a, b2a, fa, bfa,
     p1l, b1l, p2l, b2l, fl, bfl,
     p1v, b1v, p2v, b2v, fv, bfv,
     gwca, gwxa, gba2, gwcl, gwxl, gbl2, gwcv, gwxv, gbv2) = refs[:-4]
    ja_ref, jb_ref, ofa_ref, ofb_ref = refs[-4:]

    h1 = h1_ref[...]
    of = of_ref[...]
    ofa_ref[...] = of[:, :_HH]
    ofb_ref[...] = of[:, _HH:]

    def enc(h, w2, b2, g, bb):
        h = _dot(h, w2[...]) + b2[...]
        mu = jnp.mean(h, axis=-1, keepdims=True)
        var = jnp.mean((h - mu) * (h - mu), axis=-1, keepdims=True)
        return (h - mu) * lax.rsqrt(var + 1e-5) * g[...] + bb[...]

    ea = enc(h1[:, 0:256], wa2, ba2, ga, gba)
    el = enc(h1[:, 256:512], wl2, bl2, gl, gbl)
    ev = enc(h1[:, 512:768], wv2, bv2, gv, gbv)

    def projfus(x, p1, b1, p2, b2, f, bf):
        j = _dot(jnp.maximum(_dot(x, p1[...]) + b1[...], 0.0), p2[...]) + b2[...]
        return _dot(j, f[...]) + bf[...]

    d_con = projfus(of, p1c, b1c, p2c, b2c, fc, bfc)
    d_a = projfus(ea, p1a, b1a, p2a, b2a, fa, bfa)
    d_l = projfus(el, p1l, b1l, p2l, b2l, fl, bfl)
    d_v = projfus(ev, p1v, b1v, p2v, b2v, fv, bfv)

    def gate(dx, gwc, gwx, gb):
        return (jnp.sum(d_con * gwc[...], axis=-1, keepdims=True)
                + jnp.sum(dx * gwx[...], axis=-1, keepdims=True) + gb[...])

    sa = gate(d_a, gwca, gwxa, gba2)
    sl = gate(d_l, gwcl, gwxl, gbl2)
    sv = gate(d_v, gwcv, gwxv, gbv2)
    m = jnp.maximum(jnp.maximum(sa, sl), sv)
    ea_ = jnp.exp(sa - m)
    el_ = jnp.exp(sl - m)
    ev_ = jnp.exp(sv - m)
    s = ea_ + el_ + ev_
    jf = (ea_ * d_a + el_ * d_l + ev_ * d_v) / s
    ja_ref[...] = jf[:, :_HH]
    jb_ref[...] = jf[:, _HH:]


def _full_spec(a):
    nd = a.ndim
    return pl.BlockSpec(a.shape, lambda i, _nd=nd: (0,) * _nd)


def _row_spec(bn, d):
    return pl.BlockSpec((bn, d), lambda i: (i, 0))


def _half_struct():
    return jax.ShapeDtypeStruct((_N, _HH), jnp.float32)


def _tc1(h_all, of, wlist):
    in_specs = [_row_spec(_BN1, 768), _row_spec(_BN1, _H)]
    in_specs += [_full_spec(w) for w in wlist]
    return pl.pallas_call(
        _tc1_body,
        grid=(_N // _BN1,),
        in_specs=in_specs,
        out_specs=tuple(_row_spec(_BN1, _HH) for _ in range(4)),
        out_shape=tuple(_half_struct() for _ in range(4)),
    )(h_all, of, *wlist)


def _tc2_body(aja_ref, ajb_ref, aza_ref, azb_ref, dg_ref,
              wja, wjb, bj, wza, wzb, bz,
              oja_ref, ojb_ref, oza_ref, ozb_ref):
    d = jnp.maximum(dg_ref[:, 0:1], 1.0)
    r = 1.0 / d
    hj = jnp.maximum(_dot(aja_ref[...] * r, wja[...])
                     + _dot(ajb_ref[...] * r, wjb[...]) + bj[...], 0.0)
    hz = jnp.maximum(_dot(aza_ref[...] * r, wza[...])
                     + _dot(azb_ref[...] * r, wzb[...]) + bz[...], 0.0)
    oja_ref[...] = hj[:, :_HH]
    ojb_ref[...] = hj[:, _HH:]
    oza_ref[...] = hz[:, :_HH]
    ozb_ref[...] = hz[:, _HH:]


def _tc2(aggs, deg16, wj, bj, wz, bz):
    wl = [wj[:_HH], wj[_HH:], bj, wz[:_HH], wz[_HH:], bz]
    in_specs = [_row_spec(_BN2, _HH)] * 4 + [_row_spec(_BN2, 16)]
    in_specs += [_full_spec(w) for w in wl]
    return pl.pallas_call(
        _tc2_body,
        grid=(_N // _BN2,),
        in_specs=in_specs,
        out_specs=tuple(_row_spec(_BN2, _HH) for _ in range(4)),
        out_shape=tuple(_half_struct() for _ in range(4)),
    )(*aggs, deg16, *wl)


def _tc3_body(aja_ref, ajb_ref, aza_ref, azb_ref, dg_ref,
              wja, wjb, bj, wza, wzb, bz,
              p1a, p1b, b1, p2, b2, ow, ob, o_ref):
    d = jnp.maximum(dg_ref[:, 0:1], 1.0)
    r = 1.0 / d
    hj = jnp.maximum(_dot(aja_ref[...] * r, wja[...])
                     + _dot(ajb_ref[...] * r, wjb[...]) + bj[...], 0.0)
    hz = jnp.maximum(_dot(aza_ref[...] * r, wza[...])
                     + _dot(azb_ref[...] * r, wzb[...]) + bz[...], 0.0)
    h = jnp.maximum(_dot(hj, p1a[...]) + _dot(hz, p1b[...]) + b1[...], 0.0)
    h = _dot(h, p2[...]) + b2[...]
    z = _dot(h, ow[...]) + ob[...]
    m = jnp.max(z, axis=-1, keepdims=True)
    lse = m + jnp.log(jnp.sum(jnp.exp(z - m), axis=-1, keepdims=True))
    o_ref[...] = z - lse


def _tc3(aggs, deg16, wl):
    in_specs = [_row_spec(_BN2, _HH)] * 4 + [_row_spec(_BN2, 16)]
    in_specs += [_full_spec(w) for w in wl]
    return pl.pallas_call(
        _tc3_body,
        grid=(_N // _BN2,),
        in_specs=in_specs,
        out_specs=_row_spec(_BN2, _H),
        out_shape=jax.ShapeDtypeStruct((_N, _H), jnp.float32),
    )(*aggs, deg16, *wl)


def kernel(out_features, data_input, edge_index, params):
    f32 = jnp.float32
    src2 = edge_index[0].reshape(_NS, _CH, _B)
    dst2 = edge_index[1].reshape(_NS, _CH, _B)

    def b2d(b):
        return b.reshape(1, -1)

    p = params
    bbig = jnp.concatenate([p['enc_a']['l1'][1], p['enc_l']['l1'][1],
                            p['enc_v']['l1'][1]]).reshape(1, 768)

    h_all = _tc0(data_input.T, p['enc_a']['l1'][0], p['enc_l']['l1'][0],
                 p['enc_v']['l1'][0], bbig)

    def encw(name):
        e = p[name]
        return [e['l2'][0], b2d(e['l2'][1]), b2d(e['ln_g']), b2d(e['ln_b'])]

    def pfw(tag):
        return [p['proj1_' + tag][0], b2d(p['proj1_' + tag][1]),
                p['proj2_' + tag][0], b2d(p['proj2_' + tag][1]),
                p['fus_' + tag][0], b2d(p['fus_' + tag][1])]

    def gatew(tag):
        w = p['w_' + tag][0]
        return [w[:_H].reshape(1, _H), w[_H:].reshape(1, _H),
                p['w_' + tag][1].reshape(1, 1)]

    wlist = (encw('enc_a') + encw('enc_l') + encw('enc_v')
             + pfw('con') + pfw('a') + pfw('l') + pfw('v')
             + gatew('a') + gatew('l') + gatew('v'))

    ja, jb, ofa, ofb = _tc1(h_all, out_features, wlist)

    zfeat = jnp.zeros((_RPT, _HH), f32)
    zdeg = jnp.zeros((_RPT, 16), f32)
    ones_h = jnp.ones((_B, 16), f32)

    a1 = _make_agg(True)(ja, jb, ofa, ofb, src2, dst2, zfeat, ones_h, zdeg)
    deg16 = a1[4]

    h1 = _tc2(a1[:4], deg16,
              p['gcn_j1'][0], b2d(p['gcn_j1'][1]),
              p['gcn_z1'][0], b2d(p['gcn_z1'][1]))

    a2 = _make_agg(False)(*h1, src2, dst2, zfeat)

    ow = jnp.pad(p['out_layer'][0], ((0, 0), (0, _H - _C)))
    ob = jnp.concatenate([p['out_layer'][1],
                          jnp.full((_H - _C,), -1e30, f32)]).reshape(1, _H)
    p1 = p['proj1_out'][0]
    w3 = [p['gcn_j2'][0][:_HH], p['gcn_j2'][0][_HH:], b2d(p['gcn_j2'][1]),
          p['gcn_z2'][0][:_HH], p['gcn_z2'][0][_HH:], b2d(p['gcn_z2'][1]),
          p1[:_H], p1[_H:], b2d(p['proj1_out'][1]),
          p['proj2_out'][0], b2d(p['proj2_out'][1]),
          ow, ob]

    out128 = _tc3(a2, deg16, w3)
    return out128[:, :_C]


# speedup vs baseline: 8.7406x; 1.0005x over previous
import jax
import jax.numpy as jnp
from jax import lax
from jax.experimental import pallas as pl
from jax.experimental.pallas import tpu as pltpu
from jax.experimental.pallas import tpu_sc as plsc

_N = 10000
_E = 320000
_H = 128
_C = 6

_NS = 16
_B = 125
_EPS = _E // _NS
_CH = _EPS // _B
_NB = 2
_NP = 10240
_RPT = _NP // _NS
_HH = _H // 2

_BN0 = 1024
_BN1 = 1000
_BN2 = 2000


def _make_agg(with_deg):
    mesh = plsc.VectorSubcoreMesh(core_axis_name="c", subcore_axis_name="s",
                                  num_cores=2, num_subcores=_NS)
    outs = [jax.ShapeDtypeStruct((_NP, _HH), jnp.float32)
            for _ in range(4)]
    scratch = [
        pltpu.VMEM((_CH, _B), jnp.int32),
        pltpu.VMEM((_CH, _B), jnp.int32),
        pltpu.VMEM_SHARED((_NP, _HH), jnp.float32),
    ]
    scratch += [pltpu.VMEM((_B, _HH), jnp.float32) for _ in range(2 * _NB)]
    scratch += [pltpu.SemaphoreType.DMA for _ in range(5)]
    if with_deg:
        outs.append(jax.ShapeDtypeStruct((_NP, 16), jnp.float32))
        scratch += [
            pltpu.VMEM((_B, 16), jnp.float32),
            pltpu.VMEM_SHARED((_NP, 16), jnp.float32),
        ]

    def _pass(sid, xref, yref, zfeat, sidx, didx, acc, rings, sems, degp):
        ra, rb = rings
        gsa, gsb, ssa, ssb, sd = sems
        pltpu.sync_copy(zfeat, acc.at[pl.ds(sid * _RPT, _RPT)])
        if degp is not None:
            ones_h, zdeg, degy, onesv, accd = degp
            pltpu.sync_copy(zdeg, accd.at[pl.ds(sid * _RPT, _RPT)])
            pltpu.sync_copy(ones_h, onesv)
        plsc.subcore_barrier()

        def gather(i, buf, sem):
            pltpu.async_copy(xref.at[sidx.at[i]], buf, sem)

        def gwait(buf, sem):
            pltpu.make_async_copy(xref.at[sidx.at[0]], buf, sem).wait()

        for b in range(_NB):
            gather(b, ra[b], gsa)

        def superstep(base, rs, rg, gs_s, gs_g, ss_s):
            for b in range(_NB):
                gwait(rs[b], gs_s)
            for b in range(_NB):
                pltpu.async_copy(rs[b], acc.at[didx.at[base + b]], ss_s,
                                 add=True)
            if degp is not None:
                for b in range(_NB):
                    pltpu.async_copy(degp[3], degp[4].at[didx.at[base + b]],
                                     sd, add=True)
            for b in range(_NB):
                nxt = jnp.minimum(base + _NB + b, _CH - 1)
                gather(nxt, rg[b], gs_g)
            for b in range(_NB):
                pltpu.make_async_copy(rs[b], acc.at[didx.at[base + b]],
                                      ss_s).wait()
            if degp is not None:
                for b in range(_NB):
                    pltpu.make_async_copy(degp[3],
                                          degp[4].at[didx.at[base + b]],
                                          sd).wait()

        def two_steps(k, carry):
            superstep(2 * _NB * k, ra, rb, gsa, gsb, ssa)
            superstep(2 * _NB * k + _NB, rb, ra, gsb, gsa, ssb)
            return carry

        lax.fori_loop(0, _CH // (2 * _NB), two_steps, 0)
        for b in range(_NB):
            gwait(ra[b], gsa)
        plsc.subcore_barrier()
        pltpu.sync_copy(acc.at[pl.ds(sid * _RPT, _RPT)],
                        yref.at[pl.ds(sid * _RPT, _RPT)])
        if degp is not None:
            pltpu.sync_copy(degp[4].at[pl.ds(sid * _RPT, _RPT)],
                            degp[2].at[pl.ds(sid * _RPT, _RPT)])

    def _core(sid, xa, xb, ya, yb, zfeat, sidx, didx, acc, rings, sems,
              degp):
        _pass(sid, xa, ya, zfeat, sidx, didx, acc, rings, sems, degp)
        plsc.subcore_barrier()
        _pass(sid, xb, yb, zfeat, sidx, didx, acc, rings, sems, None)

    def _body_common(cid, sid, x0a, x0b, x1a, x1b, src2, dst2, zfeat,
                     y0a, y0b, y1a, y1b, sidx, didx, acc, rings, sems,
                     degp):
        pltpu.sync_copy(src2.at[sid], sidx)
        pltpu.sync_copy(dst2.at[sid], didx)

        @pl.when(cid == 0)
        def _():
            _core(sid, x0a, x0b, y0a, y0b, zfeat, sidx, didx, acc, rings,
                  sems, degp)

        @pl.when(cid == 1)
        def _():
            _core(sid, x1a, x1b, y1a, y1b, zfeat, sidx, didx, acc, rings,
                  sems, None)

    if with_deg:
        def body(x0a, x0b, x1a, x1b, src2, dst2, zfeat, ones_h, zdeg,
                 y0a, y0b, y1a, y1b, degy,
                 sidx, didx, acc, *rest):
            bufs = rest[:2 * _NB]
            sems = rest[2 * _NB:2 * _NB + 5]
            onesv, accd = rest[2 * _NB + 5:]
            cid = lax.axis_index("c")
            sid = lax.axis_index("s")
            _body_common(cid, sid, x0a, x0b, x1a, x1b, src2, dst2, zfeat,
                         y0a, y0b, y1a, y1b, sidx, didx, acc,
                         (bufs[:_NB], bufs[_NB:]), sems,
                         (ones_h, zdeg, degy, onesv, accd))
    else:
        def body(x0a, x0b, x1a, x1b, src2, dst2, zfeat,
                 y0a, y0b, y1a, y1b,
                 sidx, didx, acc, *rest):
            bufs = rest[:2 * _NB]
            sems = rest[2 * _NB:2 * _NB + 5]
            cid = lax.axis_index("c")
            sid = lax.axis_index("s")
            _body_common(cid, sid, x0a, x0b, x1a, x1b, src2, dst2, zfeat,
                         y0a, y0b, y1a, y1b, sidx, didx, acc,
                         (bufs[:_NB], bufs[_NB:]), sems, None)

    return pl.kernel(body, out_type=tuple(outs), mesh=mesh,
                     scratch_types=tuple(scratch),
                     compiler_params=pltpu.CompilerParams(
                         use_tc_tiling_on_sc=False))


def _dot(a, b):
    return jnp.dot(a, b, preferred_element_type=jnp.float32)


def _tc0_body(dt_ref, wa_ref, wl_ref, wv_ref, b_ref, o_ref):
    t = dt_ref[...]

    def blk(lo, hi, w):
        return lax.dot_general(t[lo:hi].astype(jnp.bfloat16),
                               w[...].astype(jnp.bfloat16),
                               (((0,), (0,)), ((), ())),
                               preferred_element_type=jnp.float32)

    h = jnp.concatenate([blk(0, 100, wa_ref), blk(100, 868, wl_ref),
                         blk(868, 1380, wv_ref)], axis=1)
    o_ref[...] = jnp.maximum(h + b_ref[...], 0.0)


def _tc0(dT, wa, wl, wv, bbig):
    grid = (_N + _BN0 - 1) // _BN0
    return pl.pallas_call(
        _tc0_body,
        grid=(grid,),
        in_specs=[pl.BlockSpec((1380, _BN0), lambda i: (0, i)),
                  pl.BlockSpec((100, 256), lambda i: (0, 0)),
                  pl.BlockSpec((768, 256), lambda i: (0, 0)),
                  pl.BlockSpec((512, 256), lambda i: (0, 0)),
                  pl.BlockSpec((1, 768), lambda i: (0, 0))],
        out_specs=pl.BlockSpec((_BN0, 768), lambda i: (i, 0)),
        out_shape=jax.ShapeDtypeStruct((_N, 768), jnp.float32),
    )(dT, wa, wl, wv, bbig)


def _tc1_body(h1_ref, of_ref, *refs):
    (wa2, ba2, ga, gba,
     wl2, bl2, gl, gbl,
     wv2, bv2, gv, gbv,
     p1c, b1c, p2c, b2c, fc, bfc,
     p1a, b1a, p2a, b2a, fa, bfa,
     p1l, b1l, p2l, b2l, fl, bfl,
     p1v, b1v, p2v, b2v, fv, bfv,
     gwca, gwxa, gba2, gwcl, gwxl, gbl2, gwcv, gwxv, gbv2) = refs[:-4]
    ja_ref, jb_ref, ofa_ref, ofb_ref = refs[-4:]

    h1 = h1_ref[...]
    of = of_ref[...]
    ofa_ref[...] = of[:, :_HH]
    ofb_ref[...] = of[:, _HH:]

    def enc(h, w2, b2, g, bb):
        h = _dot(h, w2[...]) + b2[...]
        mu = jnp.mean(h, axis=-1, keepdims=True)
        var = jnp.mean((h - mu) * (h - mu), axis=-1, keepdims=True)
        return (h - mu) * lax.rsqrt(var + 1e-5) * g[...] + bb[...]

    ea = enc(h1[:, 0:256], wa2, ba2, ga, gba)
    el = enc(h1[:, 256:512], wl2, bl2, gl, gbl)
    ev = enc(h1[:, 512:768], wv2, bv2, gv, gbv)

    def projfus(x, p1, b1, p2, b2, f, bf):
        j = _dot(jnp.maximum(_dot(x, p1[...]) + b1[...], 0.0), p2[...]) + b2[...]
        return _dot(j, f[...]) + bf[...]

    d_con = projfus(of, p1c, b1c, p2c, b2c, fc, bfc)
    d_a = projfus(ea, p1a, b1a, p2a, b2a, fa, bfa)
    d_l = projfus(el, p1l, b1l, p2l, b2l, fl, bfl)
    d_v = projfus(ev, p1v, b1v, p2v, b2v, fv, bfv)

    def gate(dx, gwc, gwx, gb):
        return (jnp.sum(d_con * gwc[...], axis=-1, keepdims=True)
                + jnp.sum(dx * gwx[...], axis=-1, keepdims=True) + gb[...])

    sa = gate(d_a, gwca, gwxa, gba2)
    sl = gate(d_l, gwcl, gwxl, gbl2)
    sv = gate(d_v, gwcv, gwxv, gbv2)
    m = jnp.maximum(jnp.maximum(sa, sl), sv)
    ea_ = jnp.exp(sa - m)
    el_ = jnp.exp(sl - m)
    ev_ = jnp.exp(sv - m)
    s = ea_ + el_ + ev_
    jf = (ea_ * d_a + el_ * d_l + ev_ * d_v) / s
    ja_ref[...] = jf[:, :_HH]
    jb_ref[...] = jf[:, _HH:]


def _full_spec(a):
    nd = a.ndim
    return pl.BlockSpec(a.shape, lambda i, _nd=nd: (0,) * _nd)


def _row_spec(bn, d):
    return pl.BlockSpec((bn, d), lambda i: (i, 0))


def _half_struct():
    return jax.ShapeDtypeStruct((_N, _HH), jnp.float32)


def _tc1(h_all, of, wlist):
    in_specs = [_row_spec(_BN1, 768), _row_spec(_BN1, _H)]
    in_specs += [_full_spec(w) for w in wlist]
    return pl.pallas_call(
        _tc1_body,
        grid=(_N // _BN1,),
        in_specs=in_specs,
        out_specs=tuple(_row_spec(_BN1, _HH) for _ in range(4)),
        out_shape=tuple(_half_struct() for _ in range(4)),
    )(h_all, of, *wlist)


def _tc2_body(aja_ref, ajb_ref, aza_ref, azb_ref, dg_ref,
              wja, wjb, bj, wza, wzb, bz,
              oja_ref, ojb_ref, oza_ref, ozb_ref):
    d = jnp.maximum(dg_ref[:, 0:1], 1.0)
    r = 1.0 / d
    hj = jnp.maximum(_dot(aja_ref[...] * r, wja[...])
                     + _dot(ajb_ref[...] * r, wjb[...]) + bj[...], 0.0)
    hz = jnp.maximum(_dot(aza_ref[...] * r, wza[...])
                     + _dot(azb_ref[...] * r, wzb[...]) + bz[...], 0.0)
    oja_ref[...] = hj[:, :_HH]
    ojb_ref[...] = hj[:, _HH:]
    oza_ref[...] = hz[:, :_HH]
    ozb_ref[...] = hz[:, _HH:]


def _tc2(aggs, deg16, wj, bj, wz, bz):
    wl = [wj[:_HH], wj[_HH:], bj, wz[:_HH], wz[_HH:], bz]
    in_specs = [_row_spec(_BN2, _HH)] * 4 + [_row_spec(_BN2, 16)]
    in_specs += [_full_spec(w) for w in wl]
    return pl.pallas_call(
        _tc2_body,
        grid=(_N // _BN2,),
        in_specs=in_specs,
        out_specs=tuple(_row_spec(_BN2, _HH) for _ in range(4)),
        out_shape=tuple(_half_struct() for _ in range(4)),
    )(*aggs, deg16, *wl)


def _tc3_body(aja_ref, ajb_ref, aza_ref, azb_ref, dg_ref,
              wja, wjb, bj, wza, wzb, bz,
              p1a, p1b, b1, p2, b2, ow, ob, o_ref):
    d = jnp.maximum(dg_ref[:, 0:1], 1.0)
    r = 1.0 / d
    hj = jnp.maximum(_dot(aja_ref[...] * r, wja[...])
                     + _dot(ajb_ref[...] * r, wjb[...]) + bj[...], 0.0)
    hz = jnp.maximum(_dot(aza_ref[...] * r, wza[...])
                     + _dot(azb_ref[...] * r, wzb[...]) + bz[...], 0.0)
    h = jnp.maximum(_dot(hj, p1a[...]) + _dot(hz, p1b[...]) + b1[...], 0.0)
    h = _dot(h, p2[...]) + b2[...]
    z = _dot(h, ow[...]) + ob[...]
    m = jnp.max(z, axis=-1, keepdims=True)
    lse = m + jnp.log(jnp.sum(jnp.exp(z - m), axis=-1, keepdims=True))
    o_ref[...] = z - lse


def _tc3(aggs, deg16, wl):
    in_specs = [_row_spec(_BN2, _HH)] * 4 + [_row_spec(_BN2, 16)]
    in_specs += [_full_spec(w) for w in wl]
    return pl.pallas_call(
        _tc3_body,
        grid=(_N // _BN2,),
        in_specs=in_specs,
        out_specs=_row_spec(_BN2, _H),
        out_shape=jax.ShapeDtypeStruct((_N, _H), jnp.float32),
    )(*aggs, deg16, *wl)


def kernel(out_features, data_input, edge_index, params):
    f32 = jnp.float32
    src2 = edge_index[0].reshape(_NS, _CH, _B)
    dst2 = edge_index[1].reshape(_NS, _CH, _B)

    def b2d(b):
        return b.reshape(1, -1)

    p = params
    bbig = jnp.concatenate([p['enc_a']['l1'][1], p['enc_l']['l1'][1],
                            p['enc_v']['l1'][1]]).reshape(1, 768)

    h_all = _tc0(data_input.T, p['enc_a']['l1'][0], p['enc_l']['l1'][0],
                 p['enc_v']['l1'][0], bbig)

    def encw(name):
        e = p[name]
        return [e['l2'][0], b2d(e['l2'][1]), b2d(e['ln_g']), b2d(e['ln_b'])]

    def pfw(tag):
        return [p['proj1_' + tag][0], b2d(p['proj1_' + tag][1]),
                p['proj2_' + tag][0], b2d(p['proj2_' + tag][1]),
                p['fus_' + tag][0], b2d(p['fus_' + tag][1])]

    def gatew(tag):
        w = p['w_' + tag][0]
        return [w[:_H].reshape(1, _H), w[_H:].reshape(1, _H),
                p['w_' + tag][1].reshape(1, 1)]

    wlist = (encw('enc_a') + encw('enc_l') + encw('enc_v')
             + pfw('con') + pfw('a') + pfw('l') + pfw('v')
             + gatew('a') + gatew('l') + gatew('v'))

    ja, jb, ofa, ofb = _tc1(h_all, out_features, wlist)

    zfeat = jnp.zeros((_RPT, _HH), f32)
    zdeg = jnp.zeros((_RPT, 16), f32)
    ones_h = jnp.ones((_B, 16), f32)

    a1 = _make_agg(True)(ja, jb, ofa, ofb, src2, dst2, zfeat, ones_h, zdeg)
    deg16 = a1[4]

    h1 = _tc2(a1[:4], deg16,
              p['gcn_j1'][0], b2d(p['gcn_j1'][1]),
              p['gcn_z1'][0], b2d(p['gcn_z1'][1]))

    a2 = _make_agg(False)(*h1, src2, dst2, zfeat)

    ow = jnp.pad(p['out_layer'][0], ((0, 0), (0, _H - _C)))
    ob = jnp.concatenate([p['out_layer'][1],
                          jnp.full((_H - _C,), -1e30, f32)]).reshape(1, _H)
    p1 = p['proj1_out'][0]
    w3 = [p['gcn_j2'][0][:_HH], p['gcn_j2'][0][_HH:], b2d(p['gcn_j2'][1]),
          p['gcn_z2'][0][:_HH], p['gcn_z2'][0][_HH:], b2d(p['gcn_z2'][1]),
          p1[:_H], p1[_H:], b2d(p['proj1_out'][1]),
          p['proj2_out'][0], b2d(p['proj2_out'][1]),
          ow, ob]

    out128 = _tc3(a2, deg16, w3)
    return out128[:, :_C]


# speedup vs baseline: 9.1587x; 1.0478x over previous
import jax
import jax.numpy as jnp
from jax import lax
from jax.experimental import pallas as pl
from jax.experimental.pallas import tpu as pltpu
from jax.experimental.pallas import tpu_sc as plsc

_N = 10000
_E = 320000
_H = 128
_C = 6

_NS = 16
_B = 125
_EPS = _E // _NS
_CH = _EPS // _B
_NB = 2
_NP = 10240
_RPT = _NP // _NS
_HH = _H // 2

_BN0 = 1024
_BN1 = 1024
_BN2 = 2000


def _make_agg(with_deg):
    mesh = plsc.VectorSubcoreMesh(core_axis_name="c", subcore_axis_name="s",
                                  num_cores=2, num_subcores=_NS)
    outs = [jax.ShapeDtypeStruct((_NP, _HH), jnp.float32)
            for _ in range(4)]
    scratch = [
        pltpu.VMEM((_CH, _B), jnp.int32),
        pltpu.VMEM((_CH, _B), jnp.int32),
        pltpu.VMEM_SHARED((_NP, _HH), jnp.float32),
    ]
    scratch += [pltpu.VMEM((_B, _HH), jnp.float32) for _ in range(2 * _NB)]
    scratch += [pltpu.SemaphoreType.DMA for _ in range(5)]
    if with_deg:
        outs.append(jax.ShapeDtypeStruct((_NP, 16), jnp.float32))
        scratch += [
            pltpu.VMEM((_B, 16), jnp.float32),
            pltpu.VMEM_SHARED((_NP, 16), jnp.float32),
        ]

    def _pass(sid, xref, yref, zfeat, sidx, didx, acc, rings, sems, degp):
        ra, rb = rings
        gsa, gsb, ssa, ssb, sd = sems
        pltpu.sync_copy(zfeat, acc.at[pl.ds(sid * _RPT, _RPT)])
        if degp is not None:
            ones_h, zdeg, degy, onesv, accd = degp
            pltpu.sync_copy(zdeg, accd.at[pl.ds(sid * _RPT, _RPT)])
            pltpu.sync_copy(ones_h, onesv)
        plsc.subcore_barrier()

        def gather(i, buf, sem):
            pltpu.async_copy(xref.at[sidx.at[i]], buf, sem)

        def gwait(buf, sem):
            pltpu.make_async_copy(xref.at[sidx.at[0]], buf, sem).wait()

        for b in range(_NB):
            gather(b, ra[b], gsa)

        def superstep(base, rs, rg, gs_s, gs_g, ss_s):
            for b in range(_NB):
                gwait(rs[b], gs_s)
            for b in range(_NB):
                pltpu.async_copy(rs[b], acc.at[didx.at[base + b]], ss_s,
                                 add=True)
            if degp is not None:
                for b in range(_NB):
                    pltpu.async_copy(degp[3], degp[4].at[didx.at[base + b]],
                                     sd, add=True)
            for b in range(_NB):
                nxt = jnp.minimum(base + _NB + b, _CH - 1)
                gather(nxt, rg[b], gs_g)
            for b in range(_NB):
                pltpu.make_async_copy(rs[b], acc.at[didx.at[base + b]],
                                      ss_s).wait()
            if degp is not None:
                for b in range(_NB):
                    pltpu.make_async_copy(degp[3],
                                          degp[4].at[didx.at[base + b]],
                                          sd).wait()

        def two_steps(k, carry):
            superstep(2 * _NB * k, ra, rb, gsa, gsb, ssa)
            superstep(2 * _NB * k + _NB, rb, ra, gsb, gsa, ssb)
            return carry

        lax.fori_loop(0, _CH // (2 * _NB), two_steps, 0)
        for b in range(_NB):
            gwait(ra[b], gsa)
        plsc.subcore_barrier()
        pltpu.sync_copy(acc.at[pl.ds(sid * _RPT, _RPT)],
                        yref.at[pl.ds(sid * _RPT, _RPT)])
        if degp is not None:
            pltpu.sync_copy(degp[4].at[pl.ds(sid * _RPT, _RPT)],
                            degp[2].at[pl.ds(sid * _RPT, _RPT)])

    def _core(sid, xa, xb, ya, yb, zfeat, sidx, didx, acc, rings, sems,
              degp):
        _pass(sid, xa, ya, zfeat, sidx, didx, acc, rings, sems, degp)
        plsc.subcore_barrier()
        _pass(sid, xb, yb, zfeat, sidx, didx, acc, rings, sems, None)

    def _body_common(cid, sid, x0a, x0b, x1a, x1b, src2, dst2, zfeat,
                     y0a, y0b, y1a, y1b, sidx, didx, acc, rings, sems,
                     degp):
        pltpu.sync_copy(src2.at[sid], sidx)
        pltpu.sync_copy(dst2.at[sid], didx)

        @pl.when(cid == 0)
        def _():
            _core(sid, x0a, x0b, y0a, y0b, zfeat, sidx, didx, acc, rings,
                  sems, degp)

        @pl.when(cid == 1)
        def _():
            _core(sid, x1a, x1b, y1a, y1b, zfeat, sidx, didx, acc, rings,
                  sems, None)

    if with_deg:
        def body(x0a, x0b, x1a, x1b, src2, dst2, zfeat, ones_h, zdeg,
                 y0a, y0b, y1a, y1b, degy,
                 sidx, didx, acc, *rest):
            bufs = rest[:2 * _NB]
            sems = rest[2 * _NB:2 * _NB + 5]
            onesv, accd = rest[2 * _NB + 5:]
            cid = lax.axis_index("c")
            sid = lax.axis_index("s")
            _body_common(cid, sid, x0a, x0b, x1a, x1b, src2, dst2, zfeat,
                         y0a, y0b, y1a, y1b, sidx, didx, acc,
                         (bufs[:_NB], bufs[_NB:]), sems,
                         (ones_h, zdeg, degy, onesv, accd))
    else:
        def body(x0a, x0b, x1a, x1b, src2, dst2, zfeat,
                 y0a, y0b, y1a, y1b,
                 sidx, didx, acc, *rest):
            bufs = rest[:2 * _NB]
            sems = rest[2 * _NB:2 * _NB + 5]
            cid = lax.axis_index("c")
            sid = lax.axis_index("s")
            _body_common(cid, sid, x0a, x0b, x1a, x1b, src2, dst2, zfeat,
                         y0a, y0b, y1a, y1b, sidx, didx, acc,
                         (bufs[:_NB], bufs[_NB:]), sems, None)

    return pl.kernel(body, out_type=tuple(outs), mesh=mesh,
                     scratch_types=tuple(scratch),
                     compiler_params=pltpu.CompilerParams(
                         use_tc_tiling_on_sc=False))


def _dot(a, b):
    return jnp.dot(a, b, preferred_element_type=jnp.float32)


def _tc1_body(dt_ref, of_ref, *refs):
    wa1, wl1, wv1, b1_ = refs[:4]
    refs = refs[4:]
    (wa2, ba2, ga, gba,
     wl2, bl2, gl, gbl,
     wv2, bv2, gv, gbv,
     p1c, b1c, p2c, b2c, fc, bfc,
     p1a, b1a, p2a, b2a, fa, bfa,
     p1l, b1l, p2l, b2l, fl, bfl,
     p1v, b1v, p2v, b2v, fv, bfv,
     gwca, gwxa, gba2, gwcl, gwxl, gbl2, gwcv, gwxv, gbv2) = refs[:-4]
    ja_ref, jb_ref, ofa_ref, ofb_ref = refs[-4:]

    t = dt_ref[...]

    def blk(lo, hi, w):
        return lax.dot_general(t[lo:hi].astype(jnp.bfloat16),
                               w[...].astype(jnp.bfloat16),
                               (((0,), (0,)), ((), ())),
                               preferred_element_type=jnp.float32)

    h1 = jnp.maximum(
        jnp.concatenate([blk(0, 100, wa1), blk(100, 868, wl1),
                         blk(868, 1380, wv1)], axis=1) + b1_[...], 0.0)
    of = of_ref[...]
    ofa_ref[...] = of[:, :_HH]
    ofb_ref[...] = of[:, _HH:]

    def enc(h, w2, b2, g, bb):
        h = _dot(h, w2[...]) + b2[...]
        mu = jnp.mean(h, axis=-1, keepdims=True)
        var = jnp.mean((h - mu) * (h - mu), axis=-1, keepdims=True)
        return (h - mu) * lax.rsqrt(var + 1e-5) * g[...] + bb[...]

    ea = enc(h1[:, 0:256], wa2, ba2, ga, gba)
    el = enc(h1[:, 256:512], wl2, bl2, gl, gbl)
    ev = enc(h1[:, 512:768], wv2, bv2, gv, gbv)

    def projfus(x, p1, b1, p2, b2, f, bf):
        j = _dot(jnp.maximum(_dot(x, p1[...]) + b1[...], 0.0), p2[...]) + b2[...]
        return _dot(j, f[...]) + bf[...]

    d_con = projfus(of, p1c, b1c, p2c, b2c, fc, bfc)
    d_a = projfus(ea, p1a, b1a, p2a, b2a, fa, bfa)
    d_l = projfus(el, p1l, b1l, p2l, b2l, fl, bfl)
    d_v = projfus(ev, p1v, b1v, p2v, b2v, fv, bfv)

    def gate(dx, gwc, gwx, gb):
        return (jnp.sum(d_con * gwc[...], axis=-1, keepdims=True)
                + jnp.sum(dx * gwx[...], axis=-1, keepdims=True) + gb[...])

    sa = gate(d_a, gwca, gwxa, gba2)
    sl = gate(d_l, gwcl, gwxl, gbl2)
    sv = gate(d_v, gwcv, gwxv, gbv2)
    m = jnp.maximum(jnp.maximum(sa, sl), sv)
    ea_ = jnp.exp(sa - m)
    el_ = jnp.exp(sl - m)
    ev_ = jnp.exp(sv - m)
    s = ea_ + el_ + ev_
    jf = (ea_ * d_a + el_ * d_l + ev_ * d_v) / s
    ja_ref[...] = jf[:, :_HH]
    jb_ref[...] = jf[:, _HH:]


def _full_spec(a):
    nd = a.ndim
    return pl.BlockSpec(a.shape, lambda i, _nd=nd: (0,) * _nd)


def _row_spec(bn, d):
    return pl.BlockSpec((bn, d), lambda i: (i, 0))


def _half_struct():
    return jax.ShapeDtypeStruct((_N, _HH), jnp.float32)


def _tc1(dT, of, wlist):
    in_specs = [pl.BlockSpec((1380, _BN1), lambda i: (0, i)),
                _row_spec(_BN1, _H)]
    in_specs += [_full_spec(w) for w in wlist]
    return pl.pallas_call(
        _tc1_body,
        grid=(_N // _BN1,),
        in_specs=in_specs,
        out_specs=tuple(_row_spec(_BN1, _HH) for _ in range(4)),
        out_shape=tuple(_half_struct() for _ in range(4)),
    )(dT, of, *wlist)


def _tc2_body(aja_ref, ajb_ref, aza_ref, azb_ref, dg_ref,
              wja, wjb, bj, wza, wzb, bz,
              oja_ref, ojb_ref, oza_ref, ozb_ref):
    d = jnp.maximum(dg_ref[:, 0:1], 1.0)
    r = 1.0 / d
    hj = jnp.maximum(_dot(aja_ref[...] * r, wja[...])
                     + _dot(ajb_ref[...] * r, wjb[...]) + bj[...], 0.0)
    hz = jnp.maximum(_dot(aza_ref[...] * r, wza[...])
                     + _dot(azb_ref[...] * r, wzb[...]) + bz[...], 0.0)
    oja_ref[...] = hj[:, :_HH]
    ojb_ref[...] = hj[:, _HH:]
    oza_ref[...] = hz[:, :_HH]
    ozb_ref[...] = hz[:, _HH:]


def _tc2(aggs, deg16, wj, bj, wz, bz):
    wl = [wj[:_HH], wj[_HH:], bj, wz[:_HH], wz[_HH:], bz]
    in_specs = [_row_spec(_BN2, _HH)] * 4 + [_row_spec(_BN2, 16)]
    in_specs += [_full_spec(w) for w in wl]
    return pl.pallas_call(
        _tc2_body,
        grid=(_N // _BN2,),
        in_specs=in_specs,
        out_specs=tuple(_row_spec(_BN2, _HH) for _ in range(4)),
        out_shape=tuple(_half_struct() for _ in range(4)),
    )(*aggs, deg16, *wl)


def _tc3_body(aja_ref, ajb_ref, aza_ref, azb_ref, dg_ref,
              wja, wjb, bj, wza, wzb, bz,
              p1a, p1b, b1, p2, b2, ow, ob, o_ref):
    d = jnp.maximum(dg_ref[:, 0:1], 1.0)
    r = 1.0 / d
    hj = jnp.maximum(_dot(aja_ref[...] * r, wja[...])
                     + _dot(ajb_ref[...] * r, wjb[...]) + bj[...], 0.0)
    hz = jnp.maximum(_dot(aza_ref[...] * r, wza[...])
                     + _dot(azb_ref[...] * r, wzb[...]) + bz[...], 0.0)
    h = jnp.maximum(_dot(hj, p1a[...]) + _dot(hz, p1b[...]) + b1[...], 0.0)
    h = _dot(h, p2[...]) + b2[...]
    z = _dot(h, ow[...]) + ob[...]
    m = jnp.max(z, axis=-1, keepdims=True)
    lse = m + jnp.log(jnp.sum(jnp.exp(z - m), axis=-1, keepdims=True))
    o_ref[...] = z - lse


def _tc3(aggs, deg16, wl):
    in_specs = [_row_spec(_BN2, _HH)] * 4 + [_row_spec(_BN2, 16)]
    in_specs += [_full_spec(w) for w in wl]
    return pl.pallas_call(
        _tc3_body,
        grid=(_N // _BN2,),
        in_specs=in_specs,
        out_specs=_row_spec(_BN2, _H),
        out_shape=jax.ShapeDtypeStruct((_N, _H), jnp.float32),
    )(*aggs, deg16, *wl)


def kernel(out_features, data_input, edge_index, params):
    f32 = jnp.float32
    src2 = edge_index[0].reshape(_NS, _CH, _B)
    dst2 = edge_index[1].reshape(_NS, _CH, _B)

    def b2d(b):
        return b.reshape(1, -1)

    p = params
    bbig = jnp.concatenate([p['enc_a']['l1'][1], p['enc_l']['l1'][1],
                            p['enc_v']['l1'][1]]).reshape(1, 768)
    w1list = [p['enc_a']['l1'][0], p['enc_l']['l1'][0],
              p['enc_v']['l1'][0], bbig]

    def encw(name):
        e = p[name]
        return [e['l2'][0], b2d(e['l2'][1]), b2d(e['ln_g']), b2d(e['ln_b'])]

    def pfw(tag):
        return [p['proj1_' + tag][0], b2d(p['proj1_' + tag][1]),
                p['proj2_' + tag][0], b2d(p['proj2_' + tag][1]),
                p['fus_' + tag][0], b2d(p['fus_' + tag][1])]

    def gatew(tag):
        w = p['w_' + tag][0]
        return [w[:_H].reshape(1, _H), w[_H:].reshape(1, _H),
                p['w_' + tag][1].reshape(1, 1)]

    wlist = (w1list + encw('enc_a') + encw('enc_l') + encw('enc_v')
             + pfw('con') + pfw('a') + pfw('l') + pfw('v')
             + gatew('a') + gatew('l') + gatew('v'))

    ja, jb, ofa, ofb = _tc1(data_input.T, out_features, wlist)

    zfeat = jnp.zeros((_RPT, _HH), f32)
    zdeg = jnp.zeros((_RPT, 16), f32)
    ones_h = jnp.ones((_B, 16), f32)

    a1 = _make_agg(True)(ja, jb, ofa, ofb, src2, dst2, zfeat, ones_h, zdeg)
    deg16 = a1[4]

    h1 = _tc2(a1[:4], deg16,
              p['gcn_j1'][0], b2d(p['gcn_j1'][1]),
              p['gcn_z1'][0], b2d(p['gcn_z1'][1]))

    a2 = _make_agg(False)(*h1, src2, dst2, zfeat)

    ow = jnp.pad(p['out_layer'][0], ((0, 0), (0, _H - _C)))
    ob = jnp.concatenate([p['out_layer'][1],
                          jnp.full((_H - _C,), -1e30, f32)]).reshape(1, _H)
    p1 = p['proj1_out'][0]
    w3 = [p['gcn_j2'][0][:_HH], p['gcn_j2'][0][_HH:], b2d(p['gcn_j2'][1]),
          p['gcn_z2'][0][:_HH], p['gcn_z2'][0][_HH:], b2d(p['gcn_z2'][1]),
          p1[:_H], p1[_H:], b2d(p['proj1_out'][1]),
          p['proj2_out'][0], b2d(p['proj2_out'][1]),
          ow, ob]

    out128 = _tc3(a2, deg16, w3)
    return out128[:, :_C]


# speedup vs baseline: 9.2261x; 1.0074x over previous
import jax
import jax.numpy as jnp
from jax import lax
from jax.experimental import pallas as pl
from jax.experimental.pallas import tpu as pltpu
from jax.experimental.pallas import tpu_sc as plsc

_N = 10000
_E = 320000
_H = 128
_C = 6

_NS = 16
_B = 125
_EPS = _E // _NS
_CH = _EPS // _B
_NB = 2
_NP = 10240
_RPT = _NP // _NS
_HH = _H // 2

_BN0 = 1024
_BN1 = 1024
_BN2 = 2000


def _pass(sid, xref, yref, zfeat, sidx, didx, acc, rings, sems, degp):
    ra, rb = rings
    gsa, gsb, ssa, ssb, sd = sems
    pltpu.sync_copy(zfeat, acc.at[pl.ds(sid * _RPT, _RPT)])
    if degp is not None:
        ones_h, zdeg, degy, onesv, accd = degp
        pltpu.sync_copy(zdeg, accd.at[pl.ds(sid * _RPT, _RPT)])
        pltpu.sync_copy(ones_h, onesv)
    plsc.subcore_barrier()

    def gather(i, buf, sem):
        pltpu.async_copy(xref.at[sidx.at[i]], buf, sem)

    def gwait(buf, sem):
        pltpu.make_async_copy(xref.at[sidx.at[0]], buf, sem).wait()

    for b in range(_NB):
        gather(b, ra[b], gsa)

    def superstep(base, rs, rg, gs_s, gs_g, ss_s):
        for b in range(_NB):
            gwait(rs[b], gs_s)
        for b in range(_NB):
            pltpu.async_copy(rs[b], acc.at[didx.at[base + b]], ss_s,
                             add=True)
        if degp is not None:
            for b in range(_NB):
                pltpu.async_copy(degp[3], degp[4].at[didx.at[base + b]],
                                 sd, add=True)
        for b in range(_NB):
            nxt = jnp.minimum(base + _NB + b, _CH - 1)
            gather(nxt, rg[b], gs_g)
        for b in range(_NB):
            pltpu.make_async_copy(rs[b], acc.at[didx.at[base + b]],
                                  ss_s).wait()
        if degp is not None:
            for b in range(_NB):
                pltpu.make_async_copy(degp[3],
                                      degp[4].at[didx.at[base + b]],
                                      sd).wait()

    def two_steps(k, carry):
        superstep(2 * _NB * k, ra, rb, gsa, gsb, ssa)
        superstep(2 * _NB * k + _NB, rb, ra, gsb, gsa, ssb)
        return carry

    lax.fori_loop(0, _CH // (2 * _NB), two_steps, 0)
    for b in range(_NB):
        gwait(ra[b], gsa)
    plsc.subcore_barrier()
    pltpu.sync_copy(acc.at[pl.ds(sid * _RPT, _RPT)],
                    yref.at[pl.ds(sid * _RPT, _RPT)])
    if degp is not None:
        pltpu.sync_copy(degp[4].at[pl.ds(sid * _RPT, _RPT)],
                        degp[2].at[pl.ds(sid * _RPT, _RPT)])


def _make_agg2():
    mesh = plsc.VectorSubcoreMesh(core_axis_name="c", subcore_axis_name="s",
                                  num_cores=2, num_subcores=_NS)
    outs = [jax.ShapeDtypeStruct((_NP, _HH), jnp.float32)
            for _ in range(4)]
    scratch = [
        pltpu.VMEM((_CH, _B), jnp.int32),
        pltpu.VMEM((_CH, _B), jnp.int32),
        pltpu.VMEM_SHARED((_NP, _HH), jnp.float32),
    ]
    scratch += [pltpu.VMEM((_B, _HH), jnp.float32) for _ in range(2 * _NB)]
    scratch += [pltpu.SemaphoreType.DMA for _ in range(5)]

    def _core(sid, xa, xb, ya, yb, zfeat, sidx, didx, acc, rings, sems):
        _pass(sid, xa, ya, zfeat, sidx, didx, acc, rings, sems, None)
        plsc.subcore_barrier()
        _pass(sid, xb, yb, zfeat, sidx, didx, acc, rings, sems, None)

    def body(x0a, x0b, x1a, x1b, src2, dst2, zfeat,
             y0a, y0b, y1a, y1b,
             sidx, didx, acc, *rest):
        bufs = rest[:2 * _NB]
        sems = rest[2 * _NB:2 * _NB + 5]
        rings = (bufs[:_NB], bufs[_NB:])
        cid = lax.axis_index("c")
        sid = lax.axis_index("s")
        pltpu.sync_copy(src2.at[sid], sidx)
        pltpu.sync_copy(dst2.at[sid], didx)

        @pl.when(cid == 0)
        def _():
            _core(sid, x0a, x0b, y0a, y0b, zfeat, sidx, didx, acc, rings,
                  sems)

        @pl.when(cid == 1)
        def _():
            _core(sid, x1a, x1b, y1a, y1b, zfeat, sidx, didx, acc, rings,
                  sems)

    return pl.kernel(body, out_type=tuple(outs), mesh=mesh,
                     scratch_types=tuple(scratch),
                     compiler_params=pltpu.CompilerParams(
                         use_tc_tiling_on_sc=False))


def _make_agg1(with_deg):
    mesh = plsc.VectorSubcoreMesh(core_axis_name="c", subcore_axis_name="s",
                                  num_cores=2, num_subcores=_NS)
    outs = [jax.ShapeDtypeStruct((_NP, _HH), jnp.float32) for _ in range(2)]
    scratch = [
        pltpu.VMEM((_CH, _B), jnp.int32),
        pltpu.VMEM((_CH, _B), jnp.int32),
        pltpu.VMEM_SHARED((_NP, _HH), jnp.float32),
    ]
    scratch += [pltpu.VMEM((_B, _HH), jnp.float32) for _ in range(2 * _NB)]
    scratch += [pltpu.SemaphoreType.DMA for _ in range(5)]
    if with_deg:
        outs.append(jax.ShapeDtypeStruct((_NP, 16), jnp.float32))
        scratch += [
            pltpu.VMEM((_B, 16), jnp.float32),
            pltpu.VMEM_SHARED((_NP, 16), jnp.float32),
        ]

    def _body1(xa, xb, src2, dst2, zfeat, *rest):
        if with_deg:
            ones_h, zdeg, ya, yb, degy = rest[:5]
            rest = rest[5:]
        else:
            ya, yb = rest[:2]
            rest = rest[2:]
        sidx, didx, acc = rest[:3]
        bufs = rest[3:3 + 2 * _NB]
        sems = rest[3 + 2 * _NB:3 + 2 * _NB + 5]
        rings = (bufs[:_NB], bufs[_NB:])
        if with_deg:
            onesv = rest[3 + 2 * _NB + 5]
            accd = rest[3 + 2 * _NB + 6]
            degp = (ones_h, zdeg, degy, onesv, accd)
        else:
            degp = None
        cid = lax.axis_index("c")
        sid = lax.axis_index("s")
        pltpu.sync_copy(src2.at[sid], sidx)
        pltpu.sync_copy(dst2.at[sid], didx)

        @pl.when(cid == 0)
        def _():
            _pass(sid, xa, ya, zfeat, sidx, didx, acc, rings, sems, degp)

        @pl.when(cid == 1)
        def _():
            _pass(sid, xb, yb, zfeat, sidx, didx, acc, rings, sems, None)

    return pl.kernel(_body1, out_type=tuple(outs), mesh=mesh,
                     scratch_types=tuple(scratch),
                     compiler_params=pltpu.CompilerParams(
                         use_tc_tiling_on_sc=False))


def _dot(a, b):
    return jnp.dot(a, b, preferred_element_type=jnp.float32)


def _tc1_body(dt_ref, of_ref, *refs):
    wa1, wl1, wv1, b1_ = refs[:4]
    refs = refs[4:]
    (wa2, ba2, ga, gba,
     wl2, bl2, gl, gbl,
     wv2, bv2, gv, gbv,
     p1c, b1c, p2c, b2c, fc, bfc,
     p1a, b1a, p2a, b2a, fa, bfa,
     p1l, b1l, p2l, b2l, fl, bfl,
     p1v, b1v, p2v, b2v, fv, bfv,
     gwca, gwxa, gba2, gwcl, gwxl, gbl2, gwcv, gwxv, gbv2) = refs[:-2]
    ja_ref, jb_ref = refs[-2:]

    t = dt_ref[...]

    def blk(lo, hi, w):
        return lax.dot_general(t[lo:hi].astype(jnp.bfloat16),
                               w[...].astype(jnp.bfloat16),
                               (((0,), (0,)), ((), ())),
                               preferred_element_type=jnp.float32)

    h1 = jnp.maximum(
        jnp.concatenate([blk(0, 100, wa1), blk(100, 868, wl1),
                         blk(868, 1380, wv1)], axis=1) + b1_[...], 0.0)
    of = of_ref[...]

    def enc(h, w2, b2, g, bb):
        h = _dot(h, w2[...]) + b2[...]
        mu = jnp.mean(h, axis=-1, keepdims=True)
        var = jnp.mean((h - mu) * (h - mu), axis=-1, keepdims=True)
        return (h - mu) * lax.rsqrt(var + 1e-5) * g[...] + bb[...]

    ea = enc(h1[:, 0:256], wa2, ba2, ga, gba)
    el = enc(h1[:, 256:512], wl2, bl2, gl, gbl)
    ev = enc(h1[:, 512:768], wv2, bv2, gv, gbv)

    def projfus(x, p1, b1, p2, b2, f, bf):
        j = _dot(jnp.maximum(_dot(x, p1[...]) + b1[...], 0.0), p2[...]) + b2[...]
        return _dot(j, f[...]) + bf[...]

    d_con = projfus(of, p1c, b1c, p2c, b2c, fc, bfc)
    d_a = projfus(ea, p1a, b1a, p2a, b2a, fa, bfa)
    d_l = projfus(el, p1l, b1l, p2l, b2l, fl, bfl)
    d_v = projfus(ev, p1v, b1v, p2v, b2v, fv, bfv)

    def gate(dx, gwc, gwx, gb):
        return (jnp.sum(d_con * gwc[...], axis=-1, keepdims=True)
                + jnp.sum(dx * gwx[...], axis=-1, keepdims=True) + gb[...])

    sa = gate(d_a, gwca, gwxa, gba2)
    sl = gate(d_l, gwcl, gwxl, gbl2)
    sv = gate(d_v, gwcv, gwxv, gbv2)
    m = jnp.maximum(jnp.maximum(sa, sl), sv)
    ea_ = jnp.exp(sa - m)
    el_ = jnp.exp(sl - m)
    ev_ = jnp.exp(sv - m)
    s = ea_ + el_ + ev_
    jf = (ea_ * d_a + el_ * d_l + ev_ * d_v) / s
    ja_ref[...] = jf[:, :_HH]
    jb_ref[...] = jf[:, _HH:]


def _full_spec(a):
    nd = a.ndim
    return pl.BlockSpec(a.shape, lambda i, _nd=nd: (0,) * _nd)


def _row_spec(bn, d):
    return pl.BlockSpec((bn, d), lambda i: (i, 0))


def _half_struct():
    return jax.ShapeDtypeStruct((_N, _HH), jnp.float32)


def _tc1(dT, of, wlist):
    in_specs = [pl.BlockSpec((1380, _BN1), lambda i: (0, i)),
                _row_spec(_BN1, _H)]
    in_specs += [_full_spec(w) for w in wlist]
    return pl.pallas_call(
        _tc1_body,
        grid=(_N // _BN1,),
        in_specs=in_specs,
        out_specs=tuple(_row_spec(_BN1, _HH) for _ in range(2)),
        out_shape=tuple(_half_struct() for _ in range(2)),
    )(dT, of, *wlist)


def _tc2_body(aja_ref, ajb_ref, aza_ref, azb_ref, dg_ref,
              wja, wjb, bj, wza, wzb, bz,
              oja_ref, ojb_ref, oza_ref, ozb_ref):
    d = jnp.maximum(dg_ref[:, 0:1], 1.0)
    r = 1.0 / d
    hj = jnp.maximum(_dot(aja_ref[...] * r, wja[...])
                     + _dot(ajb_ref[...] * r, wjb[...]) + bj[...], 0.0)
    hz = jnp.maximum(_dot(aza_ref[...] * r, wza[...])
                     + _dot(azb_ref[...] * r, wzb[...]) + bz[...], 0.0)
    oja_ref[...] = hj[:, :_HH]
    ojb_ref[...] = hj[:, _HH:]
    oza_ref[...] = hz[:, :_HH]
    ozb_ref[...] = hz[:, _HH:]


def _tc2(aggs, deg16, wj, bj, wz, bz):
    wl = [wj[:_HH], wj[_HH:], bj, wz[:_HH], wz[_HH:], bz]
    in_specs = [_row_spec(_BN2, _HH)] * 4 + [_row_spec(_BN2, 16)]
    in_specs += [_full_spec(w) for w in wl]
    return pl.pallas_call(
        _tc2_body,
        grid=(_N // _BN2,),
        in_specs=in_specs,
        out_specs=tuple(_row_spec(_BN2, _HH) for _ in range(4)),
        out_shape=tuple(_half_struct() for _ in range(4)),
    )(*aggs, deg16, *wl)


def _tc3_body(aja_ref, ajb_ref, aza_ref, azb_ref, dg_ref,
              wja, wjb, bj, wza, wzb, bz,
              p1a, p1b, b1, p2, b2, ow, ob, o_ref):
    d = jnp.maximum(dg_ref[:, 0:1], 1.0)
    r = 1.0 / d
    hj = jnp.maximum(_dot(aja_ref[...] * r, wja[...])
                     + _dot(ajb_ref[...] * r, wjb[...]) + bj[...], 0.0)
    hz = jnp.maximum(_dot(aza_ref[...] * r, wza[...])
                     + _dot(azb_ref[...] * r, wzb[...]) + bz[...], 0.0)
    h = jnp.maximum(_dot(hj, p1a[...]) + _dot(hz, p1b[...]) + b1[...], 0.0)
    h = _dot(h, p2[...]) + b2[...]
    z = _dot(h, ow[...]) + ob[...]
    m = jnp.max(z, axis=-1, keepdims=True)
    lse = m + jnp.log(jnp.sum(jnp.exp(z - m), axis=-1, keepdims=True))
    o_ref[...] = z - lse


def _tc3(aggs, deg16, wl):
    in_specs = [_row_spec(_BN2, _HH)] * 4 + [_row_spec(_BN2, 16)]
    in_specs += [_full_spec(w) for w in wl]
    return pl.pallas_call(
        _tc3_body,
        grid=(_N // _BN2,),
        in_specs=in_specs,
        out_specs=_row_spec(_BN2, _H),
        out_shape=jax.ShapeDtypeStruct((_N, _H), jnp.float32),
    )(*aggs, deg16, *wl)


def kernel(out_features, data_input, edge_index, params):
    f32 = jnp.float32
    src2 = edge_index[0].reshape(_NS, _CH, _B)
    dst2 = edge_index[1].reshape(_NS, _CH, _B)

    def b2d(b):
        return b.reshape(1, -1)

    p = params
    bbig = jnp.concatenate([p['enc_a']['l1'][1], p['enc_l']['l1'][1],
                            p['enc_v']['l1'][1]]).reshape(1, 768)
    w1list = [p['enc_a']['l1'][0], p['enc_l']['l1'][0],
              p['enc_v']['l1'][0], bbig]

    def encw(name):
        e = p[name]
        return [e['l2'][0], b2d(e['l2'][1]), b2d(e['ln_g']), b2d(e['ln_b'])]

    def pfw(tag):
        return [p['proj1_' + tag][0], b2d(p['proj1_' + tag][1]),
                p['proj2_' + tag][0], b2d(p['proj2_' + tag][1]),
                p['fus_' + tag][0], b2d(p['fus_' + tag][1])]

    def gatew(tag):
        w = p['w_' + tag][0]
        return [w[:_H].reshape(1, _H), w[_H:].reshape(1, _H),
                p['w_' + tag][1].reshape(1, 1)]

    wlist = (w1list + encw('enc_a') + encw('enc_l') + encw('enc_v')
             + pfw('con') + pfw('a') + pfw('l') + pfw('v')
             + gatew('a') + gatew('l') + gatew('v'))

    zfeat = jnp.zeros((_RPT, _HH), f32)
    zdeg = jnp.zeros((_RPT, 16), f32)
    ones_h = jnp.ones((_B, 16), f32)

    aza, azb, deg16 = _make_agg1(True)(
        out_features[:, :_HH], out_features[:, _HH:], src2, dst2,
        zfeat, ones_h, zdeg)

    ja, jb = _tc1(data_input.T, out_features, wlist)

    aja, ajb = _make_agg1(False)(ja, jb, src2, dst2, zfeat)

    h1 = _tc2((aja, ajb, aza, azb), deg16,
              p['gcn_j1'][0], b2d(p['gcn_j1'][1]),
              p['gcn_z1'][0], b2d(p['gcn_z1'][1]))

    a2 = _make_agg2()(*h1, src2, dst2, zfeat)

    ow = jnp.pad(p['out_layer'][0], ((0, 0), (0, _H - _C)))
    ob = jnp.concatenate([p['out_layer'][1],
                          jnp.full((_H - _C,), -1e30, f32)]).reshape(1, _H)
    p1 = p['proj1_out'][0]
    w3 = [p['gcn_j2'][0][:_HH], p['gcn_j2'][0][_HH:], b2d(p['gcn_j2'][1]),
          p['gcn_z2'][0][:_HH], p['gcn_z2'][0][_HH:], b2d(p['gcn_z2'][1]),
          p1[:_H], p1[_H:], b2d(p['proj1_out'][1]),
          p['proj2_out'][0], b2d(p['proj2_out'][1]),
          ow, ob]

    out128 = _tc3(a2, deg16, w3)
    return out128[:, :_C]
